# Initial kernel scaffold; baseline (speedup 1.0000x reference)
import jax
import jax.numpy as jnp
from jax.experimental import pallas as pl


def kernel(x, edge_index, batch, params):
    raise NotImplementedError("write your pallas kernel here")


# speedup vs baseline: 26.5707x; 26.5707x over previous
import functools

import jax
import jax.numpy as jnp
from jax import lax
from jax.experimental import pallas as pl
from jax.experimental.pallas import tpu as pltpu
from jax.experimental.pallas import tpu_sc as plsc

N = 10000
D = 128
H = 64
C = 2
G = 128

NP = 10240
NC, NS = 2, 16
NW = NC * NS
BLK = 512
ROWS_PER_TILE = NP // NS

_mesh = plsc.VectorSubcoreMesh(core_axis_name="c", subcore_axis_name="s")


def _epad(n_edges):
    per = NW * BLK
    return ((n_edges + per - 1) // per) * per


ESL = _epad(320000 + N)
ESG = _epad(320000)


def _deg_body(nblk, dst_hbm, out_hbm, idx_v, ones_v, zb, acc_sh, sem):
    c = lax.axis_index("c")
    s = lax.axis_index("s")
    w = c * NS + s
    for i in range(8):
        ones_v[pl.ds(i * 16, 16)] = jnp.ones((16,), jnp.float32)
    for i in range(ROWS_PER_TILE // 16):
        zb[pl.ds(i * 16, 16)] = jnp.zeros((16,), jnp.float32)
    pltpu.sync_copy(zb, acc_sh.at[pl.ds(s * ROWS_PER_TILE, ROWS_PER_TILE)])
    plsc.subcore_barrier()

    def blk(b, carry):
        r0 = (w * nblk + b) * (BLK // 128)
        pltpu.sync_copy(dst_hbm.at[pl.ds(r0, BLK // 128)], idx_v)

        def inner(j, carry2):
            pltpu.sync_copy(ones_v, acc_sh.at[idx_v.at[j]], add=True)
            return carry2

        return lax.fori_loop(0, BLK // 128, inner, carry)

    lax.fori_loop(0, nblk, blk, 0)
    plsc.subcore_barrier()
    pltpu.sync_copy(acc_sh.at[pl.ds(s * ROWS_PER_TILE, ROWS_PER_TILE)],
                    out_hbm.at[c, pl.ds(s * ROWS_PER_TILE, ROWS_PER_TILE)])


def _make_deg(n_edges_pad):
    nblk = n_edges_pad // (NW * BLK)
    return pl.kernel(
        functools.partial(_deg_body, nblk),
        out_type=jax.ShapeDtypeStruct((NC, NP), jnp.float32),
        mesh=_mesh,
        compiler_params=pltpu.CompilerParams(needs_layout_passes=False, use_tc_tiling_on_sc=False),
        scratch_types=[
            pltpu.VMEM((BLK // 128, 128), jnp.int32),
            pltpu.VMEM((128,), jnp.float32),
            pltpu.VMEM((ROWS_PER_TILE,), jnp.float32),
            pltpu.VMEM_SHARED((NP,), jnp.float32),
            pltpu.SemaphoreType.DMA,
        ],
    )


def _rowpass_body(nblk, table_hbm, src_hbm, dst_hbm, zeros_hbm, out_hbm,
                  src_v, dst_v, rows_v, acc_sh, sem):
    c = lax.axis_index("c")
    s = lax.axis_index("s")
    w = c * NS + s
    r = pl.ds(s * ROWS_PER_TILE, ROWS_PER_TILE)
    pltpu.sync_copy(zeros_hbm.at[r], acc_sh.at[r])
    plsc.subcore_barrier()

    def blk(b, carry):
        r0 = (w * nblk + b) * (BLK // 128)
        pltpu.sync_copy(src_hbm.at[pl.ds(r0, BLK // 128)], src_v)
        pltpu.sync_copy(dst_hbm.at[pl.ds(r0, BLK // 128)], dst_v)

        def inner(j, carry2):
            pltpu.async_copy(table_hbm.at[src_v.at[j]], rows_v, sem).wait()
            pltpu.sync_copy(rows_v, acc_sh.at[dst_v.at[j]], add=True)
            return carry2

        return lax.fori_loop(0, BLK // 128, inner, carry)

    lax.fori_loop(0, nblk, blk, 0)
    plsc.subcore_barrier()
    pltpu.sync_copy(acc_sh.at[r], out_hbm.at[c, r])


def _make_rowpass(n_edges_pad):
    nblk = n_edges_pad // (NW * BLK)
    return pl.kernel(
        functools.partial(_rowpass_body, nblk),
        out_type=jax.ShapeDtypeStruct((NC, NP, H), jnp.float32),
        mesh=_mesh,
        compiler_params=pltpu.CompilerParams(needs_layout_passes=False, use_tc_tiling_on_sc=False),
        scratch_types=[
            pltpu.VMEM((BLK // 128, 128), jnp.int32),
            pltpu.VMEM((BLK // 128, 128), jnp.int32),
            pltpu.VMEM((128, H), jnp.float32),
            pltpu.VMEM_SHARED((NP, H), jnp.float32),
            pltpu.SemaphoreType.DMA,
        ],
    )


def _gat_body(nblk, table_hbm, s_hbm, d_hbm, src_hbm, dst_hbm, zeros_hbm,
              out_hbm, den_hbm, src_v, dst_v, rows_v, s_t, d_t, ex_v, zb,
              acc_sh, den_sh, sem):
    c = lax.axis_index("c")
    s = lax.axis_index("s")
    w = c * NS + s
    r = pl.ds(s * ROWS_PER_TILE, ROWS_PER_TILE)
    pltpu.sync_copy(zeros_hbm.at[r], acc_sh.at[r])
    for i in range(ROWS_PER_TILE // 16):
        zb[pl.ds(i * 16, 16)] = jnp.zeros((16,), jnp.float32)
    pltpu.sync_copy(zb, den_sh.at[r])
    pltpu.sync_copy(s_hbm, s_t)
    pltpu.sync_copy(d_hbm, d_t)
    plsc.subcore_barrier()

    def blk(b, carry):
        r0 = (w * nblk + b) * (BLK // 128)
        pltpu.sync_copy(src_hbm.at[pl.ds(r0, BLK // 128)], src_v)
        pltpu.sync_copy(dst_hbm.at[pl.ds(r0, BLK // 128)], dst_v)

        def inner(j, carry2):
            pltpu.async_copy(table_hbm.at[src_v.at[j]], rows_v, sem).wait()
            for g in range(8):
                sl = pl.ds(g * 16, 16)
                sv = plsc.load_gather(s_t, [src_v[j, sl]])
                dv = plsc.load_gather(d_t, [dst_v[j, sl]])
                e = sv + dv
                e = jnp.where(e > 0, e, e * 0.2)
                ex_v[sl] = jnp.exp(e)
            for i in range(128):
                wv = plsc.load_gather(ex_v, [jnp.full((16,), i, jnp.int32)])
                for q in range(H // 16):
                    qs = pl.ds(q * 16, 16)
                    rows_v[i, qs] = rows_v[i, qs] * wv
            pltpu.sync_copy(ex_v, den_sh.at[dst_v.at[j]], add=True)
            pltpu.sync_copy(rows_v, acc_sh.at[dst_v.at[j]], add=True)
            return carry2

        return lax.fori_loop(0, BLK // 128, inner, carry)

    lax.fori_loop(0, nblk, blk, 0)
    plsc.subcore_barrier()
    pltpu.sync_copy(acc_sh.at[r], out_hbm.at[c, r])
    pltpu.sync_copy(den_sh.at[r], den_hbm.at[c, r])


def _make_gat(n_edges_pad):
    nblk = n_edges_pad // (NW * BLK)
    return pl.kernel(
        functools.partial(_gat_body, nblk),
        out_type=(jax.ShapeDtypeStruct((NC, NP, H), jnp.float32),
                  jax.ShapeDtypeStruct((NC, NP), jnp.float32)),
        mesh=_mesh,
        compiler_params=pltpu.CompilerParams(needs_layout_passes=False, use_tc_tiling_on_sc=False),
        scratch_types=[
            pltpu.VMEM((BLK // 128, 128), jnp.int32),
            pltpu.VMEM((BLK // 128, 128), jnp.int32),
            pltpu.VMEM((128, H), jnp.float32),
            pltpu.VMEM((NP,), jnp.float32),
            pltpu.VMEM((NP,), jnp.float32),
            pltpu.VMEM((128,), jnp.float32),
            pltpu.VMEM((ROWS_PER_TILE,), jnp.float32),
            pltpu.VMEM_SHARED((NP, H), jnp.float32),
            pltpu.VMEM_SHARED((NP,), jnp.float32),
            pltpu.SemaphoreType.DMA,
        ],
    )


RB = 1280
NG = NP // RB


def _tc1_body(xp_ref, degT_ref, wcat_ref, a1_ref,
              tg_ref, ta_ref, ts_ref, xr_ref, sd_ref):
    xb = xp_ref[...]
    h4 = jnp.dot(xb, wcat_ref[...], preferred_element_type=jnp.float32)
    deg = degT_ref[:, 0:1] + degT_ref[:, 1:2]
    dinv = lax.rsqrt(deg + 1.0)
    tg_ref[...] = h4[:, 0:H] * dinv
    ta = h4[:, H:2 * H]
    ta_ref[...] = ta
    ts_ref[...] = h4[:, 2 * H:3 * H]
    xr_ref[...] = h4[:, 3 * H:4 * H]
    sd_ref[...] = jnp.dot(ta, a1_ref[...], preferred_element_type=jnp.float32)


def _tc1(xp, degT, wcat, a1):
    f = pl.pallas_call(
        _tc1_body,
        grid=(NG,),
        in_specs=[
            pl.BlockSpec((RB, D), lambda i: (i, 0)),
            pl.BlockSpec((RB, 2), lambda i: (i, 0)),
            pl.BlockSpec((D, 4 * H), lambda i: (0, 0)),
            pl.BlockSpec((H, 128), lambda i: (0, 0)),
        ],
        out_specs=[
            pl.BlockSpec((RB, H), lambda i: (i, 0)),
            pl.BlockSpec((RB, H), lambda i: (i, 0)),
            pl.BlockSpec((RB, H), lambda i: (i, 0)),
            pl.BlockSpec((RB, H), lambda i: (i, 0)),
            pl.BlockSpec((RB, 128), lambda i: (i, 0)),
        ],
        out_shape=[jax.ShapeDtypeStruct((NP, H), jnp.float32)] * 4
        + [jax.ShapeDtypeStruct((NP, 128), jnp.float32)],
    )
    return f(xp, degT, wcat, a1)


def _tc2_body(accg_ref, acca_ref, accs_ref, denT_ref, degT_ref, xr1_ref,
              wg_ref, wa_ref, wslr_ref, a2_ref, b1_ref,
              tg2_ref, ta2_ref, ts2_ref, xr2_ref, sd2_ref):
    deg = degT_ref[:, 0:1] + degT_ref[:, 1:2]
    dinv = lax.rsqrt(deg + 1.0)
    den = denT_ref[:, 0:1] + denT_ref[:, 1:2]
    h1g = jnp.maximum(dinv * (accg_ref[0] + accg_ref[1]) + b1_ref[0:1, :], 0.0)
    h1a = jnp.maximum((acca_ref[0] + acca_ref[1]) / den + b1_ref[1:2, :], 0.0)
    h1s = jnp.maximum((accs_ref[0] + accs_ref[1]) / jnp.maximum(deg, 1.0)
                      + xr1_ref[...] + b1_ref[2:3, :], 0.0)
    tg2_ref[...] = dinv * jnp.dot(h1g, wg_ref[...],
                                  preferred_element_type=jnp.float32)
    ta2 = jnp.dot(h1a, wa_ref[...], preferred_element_type=jnp.float32)
    ta2_ref[...] = ta2
    hs2 = jnp.dot(h1s, wslr_ref[...], preferred_element_type=jnp.float32)
    ts2_ref[...] = hs2[:, 0:H]
    xr2_ref[...] = hs2[:, H:2 * H]
    sd2_ref[...] = jnp.dot(ta2, a2_ref[...], preferred_element_type=jnp.float32)


def _tc2(accg, acca, accs, denT, degT, xr1, wg, wa, wslr, a2, b1):
    f = pl.pallas_call(
        _tc2_body,
        grid=(NG,),
        in_specs=[
            pl.BlockSpec((NC, RB, H), lambda i: (0, i, 0)),
            pl.BlockSpec((NC, RB, H), lambda i: (0, i, 0)),
            pl.BlockSpec((NC, RB, H), lambda i: (0, i, 0)),
            pl.BlockSpec((RB, 2), lambda i: (i, 0)),
            pl.BlockSpec((RB, 2), lambda i: (i, 0)),
            pl.BlockSpec((RB, H), lambda i: (i, 0)),
            pl.BlockSpec((H, H), lambda i: (0, 0)),
            pl.BlockSpec((H, H), lambda i: (0, 0)),
            pl.BlockSpec((H, 2 * H), lambda i: (0, 0)),
            pl.BlockSpec((H, 128), lambda i: (0, 0)),
            pl.BlockSpec((3, H), lambda i: (0, 0)),
        ],
        out_specs=[
            pl.BlockSpec((RB, H), lambda i: (i, 0)),
            pl.BlockSpec((RB, H), lambda i: (i, 0)),
            pl.BlockSpec((RB, H), lambda i: (i, 0)),
            pl.BlockSpec((RB, H), lambda i: (i, 0)),
            pl.BlockSpec((RB, 128), lambda i: (i, 0)),
        ],
        out_shape=[jax.ShapeDtypeStruct((NP, H), jnp.float32)] * 4
        + [jax.ShapeDtypeStruct((NP, 128), jnp.float32)],
    )
    return f(accg, acca, accs, denT, degT, xr1, wg, wa, wslr, a2, b1)


def _tc3_body(accg_ref, acca_ref, accs_ref, denT_ref, degT_ref, xr2_ref,
              batchT_ref, b2_ref, wfc_ref, fcb_ref, out_ref,
              pooled_ref, cnt_ref):
    i = pl.program_id(0)

    @pl.when(i == 0)
    def _():
        pooled_ref[...] = jnp.zeros_like(pooled_ref)
        cnt_ref[...] = jnp.zeros_like(cnt_ref)

    deg = degT_ref[:, 0:1] + degT_ref[:, 1:2]
    dinv = lax.rsqrt(deg + 1.0)
    den = denT_ref[:, 0:1] + denT_ref[:, 1:2]
    h2g = jnp.maximum(dinv * (accg_ref[0] + accg_ref[1]) + b2_ref[0:1, :], 0.0)
    h2a = jnp.maximum((acca_ref[0] + acca_ref[1]) / den + b2_ref[1:2, :], 0.0)
    h2s = jnp.maximum((accs_ref[0] + accs_ref[1]) / jnp.maximum(deg, 1.0)
                      + xr2_ref[...] + b2_ref[2:3, :], 0.0)
    hcat = jnp.concatenate([h2g, h2a, h2s], axis=1)
    pb = (batchT_ref[...] ==
          lax.broadcasted_iota(jnp.int32, (G, RB), 0)).astype(jnp.float32)
    pooled_ref[...] += jnp.dot(pb, hcat, preferred_element_type=jnp.float32)
    cnt_ref[...] += jnp.sum(pb, axis=1, keepdims=True)

    @pl.when(i == NG - 1)
    def _():
        cnt = jnp.maximum(cnt_ref[...], 1.0)
        pool = pooled_ref[...] / cnt
        out_ref[...] = (jnp.dot(pool, wfc_ref[...],
                                preferred_element_type=jnp.float32)
                        + fcb_ref[...]) * (1.0 / 3.0)


def _tc3(accg, acca, accs, denT, degT, xr2, batchT, b2, wfc, fcb):
    f = pl.pallas_call(
        _tc3_body,
        grid=(NG,),
        in_specs=[
            pl.BlockSpec((NC, RB, H), lambda i: (0, i, 0)),
            pl.BlockSpec((NC, RB, H), lambda i: (0, i, 0)),
            pl.BlockSpec((NC, RB, H), lambda i: (0, i, 0)),
            pl.BlockSpec((RB, 2), lambda i: (i, 0)),
            pl.BlockSpec((RB, 2), lambda i: (i, 0)),
            pl.BlockSpec((RB, H), lambda i: (i, 0)),
            pl.BlockSpec((1, RB), lambda i: (0, i)),
            pl.BlockSpec((3, H), lambda i: (0, 0)),
            pl.BlockSpec((3 * H, 128), lambda i: (0, 0)),
            pl.BlockSpec((1, 128), lambda i: (0, 0)),
        ],
        out_specs=pl.BlockSpec((G, 128), lambda i: (0, 0)),
        out_shape=jax.ShapeDtypeStruct((G, 128), jnp.float32),
        scratch_shapes=[
            pltpu.VMEM((G, 3 * H), jnp.float32),
            pltpu.VMEM((G, 1), jnp.float32),
        ],
    )
    return f(accg, acca, accs, denT, degT, xr2, batchT, b2, wfc, fcb)


def kernel(x, edge_index, batch, params):
    p = params
    src = edge_index[0].astype(jnp.int32)
    dst = edge_index[1].astype(jnp.int32)
    loop = jnp.arange(N, dtype=jnp.int32)

    npad_sl = ESL - (320000 + N)
    pad_i_sl = jnp.arange(npad_sl, dtype=jnp.int32)
    src_sl = jnp.concatenate([src, loop, pad_i_sl % 64]).reshape(ESL // 128, 128)
    dst_sl = jnp.concatenate([dst, loop, N + pad_i_sl % (NP - N)]
                             ).reshape(ESL // 128, 128)
    npad_sg = ESG - 320000
    pad_i_sg = jnp.arange(npad_sg, dtype=jnp.int32)
    src_sg = jnp.concatenate([src, pad_i_sg % 64]).reshape(ESG // 128, 128)
    dst_sg = jnp.concatenate([dst, N + pad_i_sg % (NP - N)]
                             ).reshape(ESG // 128, 128)

    xp = jnp.pad(x, ((0, NP - N), (0, 0)))
    zeros = jnp.zeros((NP, H), jnp.float32)
    batchT = jnp.pad(batch.astype(jnp.int32), (0, NP - N),
                     constant_values=-1).reshape(1, NP)

    wcat1 = jnp.concatenate(
        [p['gcn_W1'], p['gat_W1'], p['sage_W1l'], p['sage_W1r']], axis=1)
    a1 = jnp.zeros((H, 128), jnp.float32)
    a1 = a1.at[:, 0].set(p['gat_as1']).at[:, 1].set(p['gat_ad1'])
    a2 = jnp.zeros((H, 128), jnp.float32)
    a2 = a2.at[:, 0].set(p['gat_as2']).at[:, 1].set(p['gat_ad2'])
    wslr2 = jnp.concatenate([p['sage_W2l'], p['sage_W2r']], axis=1)
    b1 = jnp.stack([p['gcn_b1'], p['gat_b1'], p['sage_b1']])
    b2 = jnp.stack([p['gcn_b2'], p['gat_b2'], p['sage_b2']])
    wfc = jnp.zeros((3 * H, 128), jnp.float32)
    wfc = wfc.at[0:H, 0:C].set(p['gcn_fcW'])
    wfc = wfc.at[H:2 * H, 0:C].set(p['gat_fcW'])
    wfc = wfc.at[2 * H:3 * H, 0:C].set(p['sage_fcW'])
    fcb = jnp.zeros((1, 128), jnp.float32)
    fcb = fcb.at[0, 0:C].set(p['gcn_fcb'] + p['gat_fcb'] + p['sage_fcb'])

    degp = _make_deg(ESG)(dst_sg)
    degT = degp.T

    tg1, ta1, ts1, xr1, sd1 = _tc1(xp, degT, wcat1, a1)
    s1 = sd1[:, 0] + 0.0
    d1 = sd1[:, 1] + 0.0

    rowpass_sl = _make_rowpass(ESL)
    rowpass_sg = _make_rowpass(ESG)
    gat_sl = _make_gat(ESL)

    accg1 = rowpass_sl(tg1, src_sl, dst_sl, zeros)
    acca1, den1 = gat_sl(ta1, s1, d1, src_sl, dst_sl, zeros)
    accs1 = rowpass_sg(ts1, src_sg, dst_sg, zeros)

    tg2, ta2, ts2, xr2, sd2 = _tc2(accg1, acca1, accs1, den1.T, degT, xr1,
                                   p['gcn_W2'], p['gat_W2'], wslr2, a2, b1)
    s2 = sd2[:, 0] + 0.0
    d2 = sd2[:, 1] + 0.0

    accg2 = rowpass_sl(tg2, src_sl, dst_sl, zeros)
    acca2, den2 = gat_sl(ta2, s2, d2, src_sl, dst_sl, zeros)
    accs2 = rowpass_sg(ts2, src_sg, dst_sg, zeros)

    out128 = _tc3(accg2, acca2, accs2, den2.T, degT, xr2, batchT, b2, wfc, fcb)
    return out128[:, :C]


# speedup vs baseline: 35.9848x; 1.3543x over previous
import functools

import jax
import jax.numpy as jnp
from jax import lax
from jax.experimental import pallas as pl
from jax.experimental.pallas import tpu as pltpu
from jax.experimental.pallas import tpu_sc as plsc

N = 10000
D = 128
H = 64
C = 2
G = 128

NP = 10240
NC, NS = 2, 16
NW = NC * NS
BLK = 512
ROWS_PER_TILE = NP // NS

_mesh = plsc.VectorSubcoreMesh(core_axis_name="c", subcore_axis_name="s")


def _epad(n_edges):
    per = NW * BLK
    return ((n_edges + per - 1) // per) * per


ESL = _epad(320000 + N)
ESG = _epad(320000)


def _splat(vec, lane):
    return lax.gather(
        vec, jnp.full((16, 1), lane, dtype=jnp.int32),
        lax.GatherDimensionNumbers(offset_dims=(), collapsed_slice_dims=(0,),
                                   start_index_map=(0,)),
        (1,), mode=lax.GatherScatterMode.PROMISE_IN_BOUNDS)


def _deg_body(nblk, dst_hbm, out_hbm, idx0, idx1, idx2, ones_v, zb, acc_sh,
              semi, sems):
    c = lax.axis_index("c")
    s = lax.axis_index("s")
    w = c * NS + s
    idxv = [idx0, idx1, idx2]
    for i in range(8):
        ones_v[pl.ds(i * 16, 16)] = jnp.ones((16,), jnp.float32)
    for i in range(ROWS_PER_TILE // 16):
        zb[pl.ds(i * 16, 16)] = jnp.zeros((16,), jnp.float32)
    pltpu.sync_copy(zb, acc_sh.at[pl.ds(s * ROWS_PER_TILE, ROWS_PER_TILE)])
    plsc.subcore_barrier()

    GPB = BLK // 128
    ngrp = nblk * GPB
    di = {0: pltpu.async_copy(dst_hbm.at[pl.ds(w * nblk * GPB, GPB)],
                              idxv[0], semi)}
    dsc = [None] * ngrp
    for b in range(nblk):
        di[b].wait()
        if b + 1 < nblk:
            r0 = (w * nblk + b + 1) * GPB
            di[b + 1] = pltpu.async_copy(dst_hbm.at[pl.ds(r0, GPB)],
                                         idxv[(b + 1) % 3], semi)
        for j in range(GPB):
            t = GPB * b + j
            if t >= 4:
                dsc[t - 4].wait()
            dsc[t] = pltpu.async_copy(ones_v, acc_sh.at[idxv[b % 3].at[j]],
                                      sems, add=True)
    for t in range(max(ngrp - 4, 0), ngrp):
        dsc[t].wait()
    plsc.subcore_barrier()
    pltpu.sync_copy(acc_sh.at[pl.ds(s * ROWS_PER_TILE, ROWS_PER_TILE)],
                    out_hbm.at[c, pl.ds(s * ROWS_PER_TILE, ROWS_PER_TILE)])


def _make_deg(n_edges_pad):
    nblk = n_edges_pad // (NW * BLK)
    return pl.kernel(
        functools.partial(_deg_body, nblk),
        out_type=jax.ShapeDtypeStruct((NC, NP), jnp.float32),
        mesh=_mesh,
        compiler_params=pltpu.CompilerParams(needs_layout_passes=False, use_tc_tiling_on_sc=False),
        scratch_types=[
            pltpu.VMEM((BLK // 128, 128), jnp.int32),
            pltpu.VMEM((BLK // 128, 128), jnp.int32),
            pltpu.VMEM((BLK // 128, 128), jnp.int32),
            pltpu.VMEM((128,), jnp.float32),
            pltpu.VMEM((ROWS_PER_TILE,), jnp.float32),
            pltpu.VMEM_SHARED((NP,), jnp.float32),
            pltpu.SemaphoreType.DMA,
            pltpu.SemaphoreType.DMA,
        ],
    )


def _rowpass_body(nblk, table_hbm, src_hbm, dst_hbm, zeros_hbm, out_hbm,
                  s0, s1, s2, d0, d1, d2, r0_, r1_, r2_, r3_, acc_sh,
                  semi, semg, sems):
    c = lax.axis_index("c")
    s = lax.axis_index("s")
    w = c * NS + s
    srcv = [s0, s1, s2]
    dstv = [d0, d1, d2]
    rows = [r0_, r1_, r2_, r3_]
    r = pl.ds(s * ROWS_PER_TILE, ROWS_PER_TILE)
    pltpu.sync_copy(zeros_hbm.at[r], acc_sh.at[r])
    plsc.subcore_barrier()

    GPB = BLK // 128
    ngrp = nblk * GPB
    base = w * nblk * GPB
    di = {0: (pltpu.async_copy(src_hbm.at[pl.ds(base, GPB)], srcv[0], semi),
              pltpu.async_copy(dst_hbm.at[pl.ds(base, GPB)], dstv[0], semi))}
    dg = [None] * ngrp
    dsc = [None] * ngrp

    def issue_scatter(u):
        dg[u].wait()
        dsc[u] = pltpu.async_copy(rows[u % 4],
                                  acc_sh.at[dstv[(u // GPB) % 3].at[u % GPB]],
                                  sems, add=True)

    for b in range(nblk):
        for dd in di[b]:
            dd.wait()
        if b + 1 < nblk:
            rr = base + (b + 1) * GPB
            di[b + 1] = (pltpu.async_copy(src_hbm.at[pl.ds(rr, GPB)],
                                          srcv[(b + 1) % 3], semi),
                         pltpu.async_copy(dst_hbm.at[pl.ds(rr, GPB)],
                                          dstv[(b + 1) % 3], semi))
        for j in range(GPB):
            t = GPB * b + j
            if t >= 4:
                dsc[t - 4].wait()
            dg[t] = pltpu.async_copy(table_hbm.at[srcv[b % 3].at[j]],
                                     rows[t % 4], semg)
            if t >= 2:
                issue_scatter(t - 2)
    for u in range(max(ngrp - 2, 0), ngrp):
        issue_scatter(u)
    for u in range(max(ngrp - 4, 0), ngrp):
        dsc[u].wait()
    plsc.subcore_barrier()
    pltpu.sync_copy(acc_sh.at[r], out_hbm.at[c, r])


def _make_rowpass(n_edges_pad):
    nblk = n_edges_pad // (NW * BLK)
    return pl.kernel(
        functools.partial(_rowpass_body, nblk),
        out_type=jax.ShapeDtypeStruct((NC, NP, H), jnp.float32),
        mesh=_mesh,
        compiler_params=pltpu.CompilerParams(needs_layout_passes=False, use_tc_tiling_on_sc=False),
        scratch_types=(
            [pltpu.VMEM((BLK // 128, 128), jnp.int32)] * 6
            + [pltpu.VMEM((128, H), jnp.float32)] * 4
            + [pltpu.VMEM_SHARED((NP, H), jnp.float32),
               pltpu.SemaphoreType.DMA,
               pltpu.SemaphoreType.DMA,
               pltpu.SemaphoreType.DMA]
        ),
    )


def _gat_body(nblk, table_hbm, s_hbm, d_hbm, src_hbm, dst_hbm, zeros_hbm,
              out_hbm, den_hbm, src_v, dst_v, r0_, r1_, r2_, r3_, s_t, d_t,
              ex_vs, zb, acc_sh, den_sh, semg, sems):
    c = lax.axis_index("c")
    s = lax.axis_index("s")
    w = c * NS + s
    rows = [r0_, r1_, r2_, r3_]
    r = pl.ds(s * ROWS_PER_TILE, ROWS_PER_TILE)
    pltpu.sync_copy(zeros_hbm.at[r], acc_sh.at[r])
    for i in range(ROWS_PER_TILE // 16):
        zb[pl.ds(i * 16, 16)] = jnp.zeros((16,), jnp.float32)
    pltpu.sync_copy(zb, den_sh.at[r])
    pltpu.sync_copy(s_hbm, s_t)
    pltpu.sync_copy(d_hbm, d_t)
    plsc.subcore_barrier()

    GPB = BLK // 128

    def blk(b, carry):
        rr = (w * nblk + b) * GPB
        pltpu.sync_copy(src_hbm.at[pl.ds(rr, GPB)], src_v)
        pltpu.sync_copy(dst_hbm.at[pl.ds(rr, GPB)], dst_v)
        dg = [pltpu.async_copy(table_hbm.at[src_v.at[j]], rows[j], semg)
              for j in range(GPB)]
        ds = []
        for j in range(GPB):
            dg[j].wait()
            rbuf = rows[j]

            def grp(g, c2, j=j, rbuf=rbuf):
                sl = pl.ds(g * 16, 16)
                sv = plsc.load_gather(s_t, [src_v[j, sl]])
                dv = plsc.load_gather(d_t, [dst_v[j, sl]])
                e = sv + dv
                e = jnp.where(e > 0, e, e * 0.2)
                ex = jnp.exp(e)
                ex_vs[j, sl] = ex
                for l in range(16):
                    wv = _splat(ex, l)
                    i = g * 16 + l
                    for q in range(H // 16):
                        qs = pl.ds(q * 16, 16)
                        rbuf[i, qs] = rbuf[i, qs] * wv
                return c2

            lax.fori_loop(0, 8, grp, 0)
            ds.append(pltpu.async_copy(ex_vs.at[j], den_sh.at[dst_v.at[j]],
                                       sems, add=True))
            ds.append(pltpu.async_copy(rbuf, acc_sh.at[dst_v.at[j]],
                                       sems, add=True))
        for d in ds:
            d.wait()
        return carry

    lax.fori_loop(0, nblk, blk, 0)
    plsc.subcore_barrier()
    pltpu.sync_copy(acc_sh.at[r], out_hbm.at[c, r])
    pltpu.sync_copy(den_sh.at[r], den_hbm.at[c, r])


def _make_gat(n_edges_pad):
    nblk = n_edges_pad // (NW * BLK)
    return pl.kernel(
        functools.partial(_gat_body, nblk),
        out_type=(jax.ShapeDtypeStruct((NC, NP, H), jnp.float32),
                  jax.ShapeDtypeStruct((NC, NP), jnp.float32)),
        mesh=_mesh,
        compiler_params=pltpu.CompilerParams(needs_layout_passes=False, use_tc_tiling_on_sc=False),
        scratch_types=(
            [pltpu.VMEM((BLK // 128, 128), jnp.int32)] * 2
            + [pltpu.VMEM((128, H), jnp.float32)] * 4
            + [pltpu.VMEM((NP,), jnp.float32),
               pltpu.VMEM((NP,), jnp.float32),
               pltpu.VMEM((BLK // 128, 128), jnp.float32),
               pltpu.VMEM((ROWS_PER_TILE,), jnp.float32),
               pltpu.VMEM_SHARED((NP, H), jnp.float32),
               pltpu.VMEM_SHARED((NP,), jnp.float32),
               pltpu.SemaphoreType.DMA,
               pltpu.SemaphoreType.DMA]
        ),
    )


RB = 1280
NG = NP // RB


def _tc1_body(xp_ref, degT_ref, wcat_ref, a1_ref,
              tg_ref, ta_ref, ts_ref, xr_ref, sd_ref):
    xb = xp_ref[...]
    h4 = jnp.dot(xb, wcat_ref[...], preferred_element_type=jnp.float32)
    deg = degT_ref[:, 0:1] + degT_ref[:, 1:2]
    dinv = lax.rsqrt(deg + 1.0)
    tg_ref[...] = h4[:, 0:H] * dinv
    ta = h4[:, H:2 * H]
    ta_ref[...] = ta
    ts_ref[...] = h4[:, 2 * H:3 * H]
    xr_ref[...] = h4[:, 3 * H:4 * H]
    sd_ref[...] = jnp.dot(ta, a1_ref[...], preferred_element_type=jnp.float32)


def _tc1(xp, degT, wcat, a1):
    f = pl.pallas_call(
        _tc1_body,
        grid=(NG,),
        in_specs=[
            pl.BlockSpec((RB, D), lambda i: (i, 0)),
            pl.BlockSpec((RB, 2), lambda i: (i, 0)),
            pl.BlockSpec((D, 4 * H), lambda i: (0, 0)),
            pl.BlockSpec((H, 128), lambda i: (0, 0)),
        ],
        out_specs=[
            pl.BlockSpec((RB, H), lambda i: (i, 0)),
            pl.BlockSpec((RB, H), lambda i: (i, 0)),
            pl.BlockSpec((RB, H), lambda i: (i, 0)),
            pl.BlockSpec((RB, H), lambda i: (i, 0)),
            pl.BlockSpec((RB, 128), lambda i: (i, 0)),
        ],
        out_shape=[jax.ShapeDtypeStruct((NP, H), jnp.float32)] * 4
        + [jax.ShapeDtypeStruct((NP, 128), jnp.float32)],
    )
    return f(xp, degT, wcat, a1)


def _tc2_body(accg_ref, acca_ref, accs_ref, denT_ref, degT_ref, xr1_ref,
              wg_ref, wa_ref, wslr_ref, a2_ref, b1_ref,
              tg2_ref, ta2_ref, ts2_ref, xr2_ref, sd2_ref):
    deg = degT_ref[:, 0:1] + degT_ref[:, 1:2]
    dinv = lax.rsqrt(deg + 1.0)
    den = denT_ref[:, 0:1] + denT_ref[:, 1:2]
    h1g = jnp.maximum(dinv * (accg_ref[0] + accg_ref[1]) + b1_ref[0:1, :], 0.0)
    h1a = jnp.maximum((acca_ref[0] + acca_ref[1]) / den + b1_ref[1:2, :], 0.0)
    h1s = jnp.maximum((accs_ref[0] + accs_ref[1]) / jnp.maximum(deg, 1.0)
                      + xr1_ref[...] + b1_ref[2:3, :], 0.0)
    tg2_ref[...] = dinv * jnp.dot(h1g, wg_ref[...],
                                  preferred_element_type=jnp.float32)
    ta2 = jnp.dot(h1a, wa_ref[...], preferred_element_type=jnp.float32)
    ta2_ref[...] = ta2
    hs2 = jnp.dot(h1s, wslr_ref[...], preferred_element_type=jnp.float32)
    ts2_ref[...] = hs2[:, 0:H]
    xr2_ref[...] = hs2[:, H:2 * H]
    sd2_ref[...] = jnp.dot(ta2, a2_ref[...], preferred_element_type=jnp.float32)


def _tc2(accg, acca, accs, denT, degT, xr1, wg, wa, wslr, a2, b1):
    f = pl.pallas_call(
        _tc2_body,
        grid=(NG,),
        in_specs=[
            pl.BlockSpec((NC, RB, H), lambda i: (0, i, 0)),
            pl.BlockSpec((NC, RB, H), lambda i: (0, i, 0)),
            pl.BlockSpec((NC, RB, H), lambda i: (0, i, 0)),
            pl.BlockSpec((RB, 2), lambda i: (i, 0)),
            pl.BlockSpec((RB, 2), lambda i: (i, 0)),
            pl.BlockSpec((RB, H), lambda i: (i, 0)),
            pl.BlockSpec((H, H), lambda i: (0, 0)),
            pl.BlockSpec((H, H), lambda i: (0, 0)),
            pl.BlockSpec((H, 2 * H), lambda i: (0, 0)),
            pl.BlockSpec((H, 128), lambda i: (0, 0)),
            pl.BlockSpec((3, H), lambda i: (0, 0)),
        ],
        out_specs=[
            pl.BlockSpec((RB, H), lambda i: (i, 0)),
            pl.BlockSpec((RB, H), lambda i: (i, 0)),
            pl.BlockSpec((RB, H), lambda i: (i, 0)),
            pl.BlockSpec((RB, H), lambda i: (i, 0)),
            pl.BlockSpec((RB, 128), lambda i: (i, 0)),
        ],
        out_shape=[jax.ShapeDtypeStruct((NP, H), jnp.float32)] * 4
        + [jax.ShapeDtypeStruct((NP, 128), jnp.float32)],
    )
    return f(accg, acca, accs, denT, degT, xr1, wg, wa, wslr, a2, b1)


def _tc3_body(accg_ref, acca_ref, accs_ref, denT_ref, degT_ref, xr2_ref,
              batchT_ref, b2_ref, wfc_ref, fcb_ref, out_ref,
              pooled_ref, cnt_ref):
    i = pl.program_id(0)

    @pl.when(i == 0)
    def _():
        pooled_ref[...] = jnp.zeros_like(pooled_ref)
        cnt_ref[...] = jnp.zeros_like(cnt_ref)

    deg = degT_ref[:, 0:1] + degT_ref[:, 1:2]
    dinv = lax.rsqrt(deg + 1.0)
    den = denT_ref[:, 0:1] + denT_ref[:, 1:2]
    h2g = jnp.maximum(dinv * (accg_ref[0] + accg_ref[1]) + b2_ref[0:1, :], 0.0)
    h2a = jnp.maximum((acca_ref[0] + acca_ref[1]) / den + b2_ref[1:2, :], 0.0)
    h2s = jnp.maximum((accs_ref[0] + accs_ref[1]) / jnp.maximum(deg, 1.0)
                      + xr2_ref[...] + b2_ref[2:3, :], 0.0)
    hcat = jnp.concatenate([h2g, h2a, h2s], axis=1)
    pb = (batchT_ref[...] ==
          lax.broadcasted_iota(jnp.int32, (G, RB), 0)).astype(jnp.float32)
    pooled_ref[...] += jnp.dot(pb, hcat, preferred_element_type=jnp.float32)
    cnt_ref[...] += jnp.sum(pb, axis=1, keepdims=True)

    @pl.when(i == NG - 1)
    def _():
        cnt = jnp.maximum(cnt_ref[...], 1.0)
        pool = pooled_ref[...] / cnt
        out_ref[...] = (jnp.dot(pool, wfc_ref[...],
                                preferred_element_type=jnp.float32)
                        + fcb_ref[...]) * (1.0 / 3.0)


def _tc3(accg, acca, accs, denT, degT, xr2, batchT, b2, wfc, fcb):
    f = pl.pallas_call(
        _tc3_body,
        grid=(NG,),
        in_specs=[
            pl.BlockSpec((NC, RB, H), lambda i: (0, i, 0)),
            pl.BlockSpec((NC, RB, H), lambda i: (0, i, 0)),
            pl.BlockSpec((NC, RB, H), lambda i: (0, i, 0)),
            pl.BlockSpec((RB, 2), lambda i: (i, 0)),
            pl.BlockSpec((RB, 2), lambda i: (i, 0)),
            pl.BlockSpec((RB, H), lambda i: (i, 0)),
            pl.BlockSpec((1, RB), lambda i: (0, i)),
            pl.BlockSpec((3, H), lambda i: (0, 0)),
            pl.BlockSpec((3 * H, 128), lambda i: (0, 0)),
            pl.BlockSpec((1, 128), lambda i: (0, 0)),
        ],
        out_specs=pl.BlockSpec((G, 128), lambda i: (0, 0)),
        out_shape=jax.ShapeDtypeStruct((G, 128), jnp.float32),
        scratch_shapes=[
            pltpu.VMEM((G, 3 * H), jnp.float32),
            pltpu.VMEM((G, 1), jnp.float32),
        ],
    )
    return f(accg, acca, accs, denT, degT, xr2, batchT, b2, wfc, fcb)


def kernel(x, edge_index, batch, params):
    p = params
    src = edge_index[0].astype(jnp.int32)
    dst = edge_index[1].astype(jnp.int32)
    loop = jnp.arange(N, dtype=jnp.int32)

    npad_sl = ESL - (320000 + N)
    pad_i_sl = jnp.arange(npad_sl, dtype=jnp.int32)
    src_sl = jnp.concatenate([src, loop, pad_i_sl % 64]).reshape(ESL // 128, 128)
    dst_sl = jnp.concatenate([dst, loop, N + pad_i_sl % (NP - N)]
                             ).reshape(ESL // 128, 128)
    npad_sg = ESG - 320000
    pad_i_sg = jnp.arange(npad_sg, dtype=jnp.int32)
    src_sg = jnp.concatenate([src, pad_i_sg % 64]).reshape(ESG // 128, 128)
    dst_sg = jnp.concatenate([dst, N + pad_i_sg % (NP - N)]
                             ).reshape(ESG // 128, 128)

    xp = jnp.pad(x, ((0, NP - N), (0, 0)))
    zeros = jnp.zeros((NP, H), jnp.float32)
    batchT = jnp.pad(batch.astype(jnp.int32), (0, NP - N),
                     constant_values=-1).reshape(1, NP)

    wcat1 = jnp.concatenate(
        [p['gcn_W1'], p['gat_W1'], p['sage_W1l'], p['sage_W1r']], axis=1)
    a1 = jnp.zeros((H, 128), jnp.float32)
    a1 = a1.at[:, 0].set(p['gat_as1']).at[:, 1].set(p['gat_ad1'])
    a2 = jnp.zeros((H, 128), jnp.float32)
    a2 = a2.at[:, 0].set(p['gat_as2']).at[:, 1].set(p['gat_ad2'])
    wslr2 = jnp.concatenate([p['sage_W2l'], p['sage_W2r']], axis=1)
    b1 = jnp.stack([p['gcn_b1'], p['gat_b1'], p['sage_b1']])
    b2 = jnp.stack([p['gcn_b2'], p['gat_b2'], p['sage_b2']])
    wfc = jnp.zeros((3 * H, 128), jnp.float32)
    wfc = wfc.at[0:H, 0:C].set(p['gcn_fcW'])
    wfc = wfc.at[H:2 * H, 0:C].set(p['gat_fcW'])
    wfc = wfc.at[2 * H:3 * H, 0:C].set(p['sage_fcW'])
    fcb = jnp.zeros((1, 128), jnp.float32)
    fcb = fcb.at[0, 0:C].set(p['gcn_fcb'] + p['gat_fcb'] + p['sage_fcb'])

    degp = _make_deg(ESG)(dst_sg)
    degT = degp.T

    tg1, ta1, ts1, xr1, sd1 = _tc1(xp, degT, wcat1, a1)
    s1 = sd1[:, 0] + 0.0
    d1 = sd1[:, 1] + 0.0

    rowpass_sl = _make_rowpass(ESL)
    rowpass_sg = _make_rowpass(ESG)
    gat_sl = _make_gat(ESL)

    accg1 = rowpass_sl(tg1, src_sl, dst_sl, zeros)
    acca1, den1 = gat_sl(ta1, s1, d1, src_sl, dst_sl, zeros)
    accs1 = rowpass_sg(ts1, src_sg, dst_sg, zeros)

    tg2, ta2, ts2, xr2, sd2 = _tc2(accg1, acca1, accs1, den1.T, degT, xr1,
                                   p['gcn_W2'], p['gat_W2'], wslr2, a2, b1)
    s2 = sd2[:, 0] + 0.0
    d2 = sd2[:, 1] + 0.0

    accg2 = rowpass_sl(tg2, src_sl, dst_sl, zeros)
    acca2, den2 = gat_sl(ta2, s2, d2, src_sl, dst_sl, zeros)
    accs2 = rowpass_sg(ts2, src_sg, dst_sg, zeros)

    out128 = _tc3(accg2, acca2, accs2, den2.T, degT, xr2, batchT, b2, wfc, fcb)
    return out128[:, :C]


# speedup vs baseline: 51.3025x; 1.4257x over previous
import functools

import jax
import jax.numpy as jnp
from jax import lax
from jax.experimental import pallas as pl
from jax.experimental.pallas import tpu as pltpu
from jax.experimental.pallas import tpu_sc as plsc

N = 10000
D = 128
H = 64
C = 2
G = 128

NP = 10240
NC, NS = 2, 16
NW = NC * NS
BLK = 512
ROWS_PER_TILE = NP // NS

_mesh = plsc.VectorSubcoreMesh(core_axis_name="c", subcore_axis_name="s")


def _epad(n_edges):
    per = NW * BLK
    return ((n_edges + per - 1) // per) * per


ESL = _epad(320000 + N)
ESG = _epad(320000)


def _splat(vec, lane):
    return lax.gather(
        vec, jnp.full((16, 1), lane, dtype=jnp.int32),
        lax.GatherDimensionNumbers(offset_dims=(), collapsed_slice_dims=(0,),
                                   start_index_map=(0,)),
        (1,), mode=lax.GatherScatterMode.PROMISE_IN_BOUNDS)


def _deg_body(nblk, dst_hbm, out_hbm, idx0, idx1, idx2, ones_v, zb, acc_sh,
              semi, sems):
    c = lax.axis_index("c")
    s = lax.axis_index("s")
    w = c * NS + s
    idxv = [idx0, idx1, idx2]
    for i in range(8):
        ones_v[pl.ds(i * 16, 16)] = jnp.ones((16,), jnp.float32)
    for i in range(ROWS_PER_TILE // 16):
        zb[pl.ds(i * 16, 16)] = jnp.zeros((16,), jnp.float32)
    pltpu.sync_copy(zb, acc_sh.at[pl.ds(s * ROWS_PER_TILE, ROWS_PER_TILE)])
    plsc.subcore_barrier()

    GPB = BLK // 128
    ngrp = nblk * GPB
    di = {0: pltpu.async_copy(dst_hbm.at[pl.ds(w * nblk * GPB, GPB)],
                              idxv[0], semi)}
    dsc = [None] * ngrp
    for b in range(nblk):
        di[b].wait()
        if b + 1 < nblk:
            r0 = (w * nblk + b + 1) * GPB
            di[b + 1] = pltpu.async_copy(dst_hbm.at[pl.ds(r0, GPB)],
                                         idxv[(b + 1) % 3], semi)
        for j in range(GPB):
            t = GPB * b + j
            if t >= 4:
                dsc[t - 4].wait()
            dsc[t] = pltpu.async_copy(ones_v, acc_sh.at[idxv[b % 3].at[j]],
                                      sems, add=True)
    for t in range(max(ngrp - 4, 0), ngrp):
        dsc[t].wait()
    plsc.subcore_barrier()
    pltpu.sync_copy(acc_sh.at[pl.ds(s * ROWS_PER_TILE, ROWS_PER_TILE)],
                    out_hbm.at[c, pl.ds(s * ROWS_PER_TILE, ROWS_PER_TILE)])


def _make_deg(n_edges_pad):
    nblk = n_edges_pad // (NW * BLK)
    return pl.kernel(
        functools.partial(_deg_body, nblk),
        out_type=jax.ShapeDtypeStruct((NC, NP), jnp.float32),
        mesh=_mesh,
        compiler_params=pltpu.CompilerParams(needs_layout_passes=False, use_tc_tiling_on_sc=False),
        scratch_types=[
            pltpu.VMEM((BLK // 128, 128), jnp.int32),
            pltpu.VMEM((BLK // 128, 128), jnp.int32),
            pltpu.VMEM((BLK // 128, 128), jnp.int32),
            pltpu.VMEM((128,), jnp.float32),
            pltpu.VMEM((ROWS_PER_TILE,), jnp.float32),
            pltpu.VMEM_SHARED((NP,), jnp.float32),
            pltpu.SemaphoreType.DMA,
            pltpu.SemaphoreType.DMA,
        ],
    )


def _rowpass_body(nblk, table_hbm, src_hbm, dst_hbm, zeros_hbm, out_hbm,
                  s0, s1, s2, s3, d0, d1, d2, d3,
                  r0_, r1_, r2_, r3_, r4_, r5_, r6_, r7_, acc_sh,
                  semi, semg, sems):
    c = lax.axis_index("c")
    s = lax.axis_index("s")
    w = c * NS + s
    srcv = [s0, s1, s2, s3]
    dstv = [d0, d1, d2, d3]
    rows = [r0_, r1_, r2_, r3_, r4_, r5_, r6_, r7_]
    r = pl.ds(s * ROWS_PER_TILE, ROWS_PER_TILE)
    pltpu.sync_copy(zeros_hbm.at[r], acc_sh.at[r])
    plsc.subcore_barrier()

    GPB = BLK // 128
    ngrp = nblk * GPB
    base = w * nblk * GPB
    di = {0: (pltpu.async_copy(src_hbm.at[pl.ds(base, GPB)], srcv[0], semi),
              pltpu.async_copy(dst_hbm.at[pl.ds(base, GPB)], dstv[0], semi))}
    dg = [None] * ngrp
    dsc = [None] * ngrp

    def issue_scatter(u):
        dg[u].wait()
        dsc[u] = pltpu.async_copy(rows[u % 8],
                                  acc_sh.at[dstv[(u // GPB) % 4].at[u % GPB]],
                                  sems, add=True)

    for b in range(nblk):
        for dd in di[b]:
            dd.wait()
        if b + 1 < nblk:
            rr = base + (b + 1) * GPB
            di[b + 1] = (pltpu.async_copy(src_hbm.at[pl.ds(rr, GPB)],
                                          srcv[(b + 1) % 4], semi),
                         pltpu.async_copy(dst_hbm.at[pl.ds(rr, GPB)],
                                          dstv[(b + 1) % 4], semi))
        for j in range(GPB):
            t = GPB * b + j
            if t >= 8:
                dsc[t - 8].wait()
            dg[t] = pltpu.async_copy(table_hbm.at[srcv[b % 4].at[j]],
                                     rows[t % 8], semg)
            if t >= 4:
                issue_scatter(t - 4)
    for u in range(max(ngrp - 4, 0), ngrp):
        issue_scatter(u)
    for u in range(max(ngrp - 8, 0), ngrp):
        dsc[u].wait()
    plsc.subcore_barrier()
    pltpu.sync_copy(acc_sh.at[r], out_hbm.at[c, r])


def _make_rowpass(n_edges_pad):
    nblk = n_edges_pad // (NW * BLK)
    return pl.kernel(
        functools.partial(_rowpass_body, nblk),
        out_type=jax.ShapeDtypeStruct((NC, NP, H), jnp.float32),
        mesh=_mesh,
        compiler_params=pltpu.CompilerParams(needs_layout_passes=False, use_tc_tiling_on_sc=False),
        scratch_types=(
            [pltpu.VMEM((BLK // 128, 128), jnp.int32)] * 8
            + [pltpu.VMEM((128, H), jnp.float32)] * 8
            + [pltpu.VMEM_SHARED((NP, H), jnp.float32),
               pltpu.SemaphoreType.DMA,
               pltpu.SemaphoreType.DMA,
               pltpu.SemaphoreType.DMA]
        ),
    )


def _gat_body(nblk, table_hbm, s_hbm, d_hbm, src_hbm, dst_hbm, zeros_hbm,
              out_hbm, den_hbm, src_v, dst_v, r0_, r1_, r2_, r3_, s_t, d_t,
              ex_vs, zb, acc_sh, den_sh, semg, sems):
    c = lax.axis_index("c")
    s = lax.axis_index("s")
    w = c * NS + s
    rows = [r0_, r1_, r2_, r3_]
    r = pl.ds(s * ROWS_PER_TILE, ROWS_PER_TILE)
    pltpu.sync_copy(zeros_hbm.at[r], acc_sh.at[r])
    for i in range(ROWS_PER_TILE // 16):
        zb[pl.ds(i * 16, 16)] = jnp.zeros((16,), jnp.float32)
    pltpu.sync_copy(zb, den_sh.at[r])
    pltpu.sync_copy(s_hbm, s_t)
    pltpu.sync_copy(d_hbm, d_t)
    plsc.subcore_barrier()

    GPB = BLK // 128

    def blk(b, carry):
        rr = (w * nblk + b) * GPB
        pltpu.sync_copy(src_hbm.at[pl.ds(rr, GPB)], src_v)
        pltpu.sync_copy(dst_hbm.at[pl.ds(rr, GPB)], dst_v)
        dg = [pltpu.async_copy(table_hbm.at[src_v.at[j]], rows[j], semg)
              for j in range(GPB)]
        ds = []
        for j in range(GPB):
            dg[j].wait()
            rbuf = rows[j]

            @plsc.parallel_loop(0, 8, unroll=2)
            def _grp(g, j=j, rbuf=rbuf):
                sl = pl.ds(g * 16, 16)
                sv = plsc.load_gather(s_t, [src_v[j, sl]])
                dv = plsc.load_gather(d_t, [dst_v[j, sl]])
                e = sv + dv
                e = jnp.where(e > 0, e, e * 0.2)
                ex = jnp.exp(e)
                ex_vs[j, sl] = ex
                for l in range(16):
                    wv = _splat(ex, l)
                    i = g * 16 + l
                    for q in range(H // 16):
                        qs = pl.ds(q * 16, 16)
                        rbuf[i, qs] = rbuf[i, qs] * wv
            ds.append(pltpu.async_copy(ex_vs.at[j], den_sh.at[dst_v.at[j]],
                                       sems, add=True))
            ds.append(pltpu.async_copy(rbuf, acc_sh.at[dst_v.at[j]],
                                       sems, add=True))
        for d in ds:
            d.wait()
        return carry

    lax.fori_loop(0, nblk, blk, 0)
    plsc.subcore_barrier()
    pltpu.sync_copy(acc_sh.at[r], out_hbm.at[c, r])
    pltpu.sync_copy(den_sh.at[r], den_hbm.at[c, r])


def _make_gat(n_edges_pad):
    nblk = n_edges_pad // (NW * BLK)
    return pl.kernel(
        functools.partial(_gat_body, nblk),
        out_type=(jax.ShapeDtypeStruct((NC, NP, H), jnp.float32),
                  jax.ShapeDtypeStruct((NC, NP), jnp.float32)),
        mesh=_mesh,
        compiler_params=pltpu.CompilerParams(needs_layout_passes=False, use_tc_tiling_on_sc=False),
        scratch_types=(
            [pltpu.VMEM((BLK // 128, 128), jnp.int32)] * 2
            + [pltpu.VMEM((128, H), jnp.float32)] * 4
            + [pltpu.VMEM((NP,), jnp.float32),
               pltpu.VMEM((NP,), jnp.float32),
               pltpu.VMEM((BLK // 128, 128), jnp.float32),
               pltpu.VMEM((ROWS_PER_TILE,), jnp.float32),
               pltpu.VMEM_SHARED((NP, H), jnp.float32),
               pltpu.VMEM_SHARED((NP,), jnp.float32),
               pltpu.SemaphoreType.DMA,
               pltpu.SemaphoreType.DMA]
        ),
    )


RB = 1280
NG = NP // RB


def _tc1_body(xp_ref, degT_ref, wcat_ref, a1_ref,
              tg_ref, ta_ref, ts_ref, xr_ref, sd_ref):
    xb = xp_ref[...]
    h4 = jnp.dot(xb, wcat_ref[...], preferred_element_type=jnp.float32)
    deg = degT_ref[:, 0:1] + degT_ref[:, 1:2]
    dinv = lax.rsqrt(deg + 1.0)
    tg_ref[...] = h4[:, 0:H] * dinv
    ta = h4[:, H:2 * H]
    ta_ref[...] = ta
    ts_ref[...] = h4[:, 2 * H:3 * H]
    xr_ref[...] = h4[:, 3 * H:4 * H]
    sd_ref[...] = jnp.dot(ta, a1_ref[...], preferred_element_type=jnp.float32)


def _tc1(xp, degT, wcat, a1):
    f = pl.pallas_call(
        _tc1_body,
        grid=(NG,),
        in_specs=[
            pl.BlockSpec((RB, D), lambda i: (i, 0)),
            pl.BlockSpec((RB, 2), lambda i: (i, 0)),
            pl.BlockSpec((D, 4 * H), lambda i: (0, 0)),
            pl.BlockSpec((H, 128), lambda i: (0, 0)),
        ],
        out_specs=[
            pl.BlockSpec((RB, H), lambda i: (i, 0)),
            pl.BlockSpec((RB, H), lambda i: (i, 0)),
            pl.BlockSpec((RB, H), lambda i: (i, 0)),
            pl.BlockSpec((RB, H), lambda i: (i, 0)),
            pl.BlockSpec((RB, 128), lambda i: (i, 0)),
        ],
        out_shape=[jax.ShapeDtypeStruct((NP, H), jnp.float32)] * 4
        + [jax.ShapeDtypeStruct((NP, 128), jnp.float32)],
    )
    return f(xp, degT, wcat, a1)


def _tc2_body(accg_ref, acca_ref, accs_ref, denT_ref, degT_ref, xr1_ref,
              wg_ref, wa_ref, wslr_ref, a2_ref, b1_ref,
              tg2_ref, ta2_ref, ts2_ref, xr2_ref, sd2_ref):
    deg = degT_ref[:, 0:1] + degT_ref[:, 1:2]
    dinv = lax.rsqrt(deg + 1.0)
    den = denT_ref[:, 0:1] + denT_ref[:, 1:2]
    h1g = jnp.maximum(dinv * (accg_ref[0] + accg_ref[1]) + b1_ref[0:1, :], 0.0)
    h1a = jnp.maximum((acca_ref[0] + acca_ref[1]) / den + b1_ref[1:2, :], 0.0)
    h1s = jnp.maximum((accs_ref[0] + accs_ref[1]) / jnp.maximum(deg, 1.0)
                      + xr1_ref[...] + b1_ref[2:3, :], 0.0)
    tg2_ref[...] = dinv * jnp.dot(h1g, wg_ref[...],
                                  preferred_element_type=jnp.float32)
    ta2 = jnp.dot(h1a, wa_ref[...], preferred_element_type=jnp.float32)
    ta2_ref[...] = ta2
    hs2 = jnp.dot(h1s, wslr_ref[...], preferred_element_type=jnp.float32)
    ts2_ref[...] = hs2[:, 0:H]
    xr2_ref[...] = hs2[:, H:2 * H]
    sd2_ref[...] = jnp.dot(ta2, a2_ref[...], preferred_element_type=jnp.float32)


def _tc2(accg, acca, accs, denT, degT, xr1, wg, wa, wslr, a2, b1):
    f = pl.pallas_call(
        _tc2_body,
        grid=(NG,),
        in_specs=[
            pl.BlockSpec((NC, RB, H), lambda i: (0, i, 0)),
            pl.BlockSpec((NC, RB, H), lambda i: (0, i, 0)),
            pl.BlockSpec((NC, RB, H), lambda i: (0, i, 0)),
            pl.BlockSpec((RB, 2), lambda i: (i, 0)),
            pl.BlockSpec((RB, 2), lambda i: (i, 0)),
            pl.BlockSpec((RB, H), lambda i: (i, 0)),
            pl.BlockSpec((H, H), lambda i: (0, 0)),
            pl.BlockSpec((H, H), lambda i: (0, 0)),
            pl.BlockSpec((H, 2 * H), lambda i: (0, 0)),
            pl.BlockSpec((H, 128), lambda i: (0, 0)),
            pl.BlockSpec((3, H), lambda i: (0, 0)),
        ],
        out_specs=[
            pl.BlockSpec((RB, H), lambda i: (i, 0)),
            pl.BlockSpec((RB, H), lambda i: (i, 0)),
            pl.BlockSpec((RB, H), lambda i: (i, 0)),
            pl.BlockSpec((RB, H), lambda i: (i, 0)),
            pl.BlockSpec((RB, 128), lambda i: (i, 0)),
        ],
        out_shape=[jax.ShapeDtypeStruct((NP, H), jnp.float32)] * 4
        + [jax.ShapeDtypeStruct((NP, 128), jnp.float32)],
    )
    return f(accg, acca, accs, denT, degT, xr1, wg, wa, wslr, a2, b1)


def _tc3_body(accg_ref, acca_ref, accs_ref, denT_ref, degT_ref, xr2_ref,
              batchT_ref, b2_ref, wfc_ref, fcb_ref, out_ref,
              pooled_ref, cnt_ref):
    i = pl.program_id(0)

    @pl.when(i == 0)
    def _():
        pooled_ref[...] = jnp.zeros_like(pooled_ref)
        cnt_ref[...] = jnp.zeros_like(cnt_ref)

    deg = degT_ref[:, 0:1] + degT_ref[:, 1:2]
    dinv = lax.rsqrt(deg + 1.0)
    den = denT_ref[:, 0:1] + denT_ref[:, 1:2]
    h2g = jnp.maximum(dinv * (accg_ref[0] + accg_ref[1]) + b2_ref[0:1, :], 0.0)
    h2a = jnp.maximum((acca_ref[0] + acca_ref[1]) / den + b2_ref[1:2, :], 0.0)
    h2s = jnp.maximum((accs_ref[0] + accs_ref[1]) / jnp.maximum(deg, 1.0)
                      + xr2_ref[...] + b2_ref[2:3, :], 0.0)
    hcat = jnp.concatenate([h2g, h2a, h2s], axis=1)
    pb = (batchT_ref[...] ==
          lax.broadcasted_iota(jnp.int32, (G, RB), 0)).astype(jnp.float32)
    pooled_ref[...] += jnp.dot(pb, hcat, preferred_element_type=jnp.float32)
    cnt_ref[...] += jnp.sum(pb, axis=1, keepdims=True)

    @pl.when(i == NG - 1)
    def _():
        cnt = jnp.maximum(cnt_ref[...], 1.0)
        pool = pooled_ref[...] / cnt
        out_ref[...] = (jnp.dot(pool, wfc_ref[...],
                                preferred_element_type=jnp.float32)
                        + fcb_ref[...]) * (1.0 / 3.0)


def _tc3(accg, acca, accs, denT, degT, xr2, batchT, b2, wfc, fcb):
    f = pl.pallas_call(
        _tc3_body,
        grid=(NG,),
        in_specs=[
            pl.BlockSpec((NC, RB, H), lambda i: (0, i, 0)),
            pl.BlockSpec((NC, RB, H), lambda i: (0, i, 0)),
            pl.BlockSpec((NC, RB, H), lambda i: (0, i, 0)),
            pl.BlockSpec((RB, 2), lambda i: (i, 0)),
            pl.BlockSpec((RB, 2), lambda i: (i, 0)),
            pl.BlockSpec((RB, H), lambda i: (i, 0)),
            pl.BlockSpec((1, RB), lambda i: (0, i)),
            pl.BlockSpec((3, H), lambda i: (0, 0)),
            pl.BlockSpec((3 * H, 128), lambda i: (0, 0)),
            pl.BlockSpec((1, 128), lambda i: (0, 0)),
        ],
        out_specs=pl.BlockSpec((G, 128), lambda i: (0, 0)),
        out_shape=jax.ShapeDtypeStruct((G, 128), jnp.float32),
        scratch_shapes=[
            pltpu.VMEM((G, 3 * H), jnp.float32),
            pltpu.VMEM((G, 1), jnp.float32),
        ],
    )
    return f(accg, acca, accs, denT, degT, xr2, batchT, b2, wfc, fcb)


def kernel(x, edge_index, batch, params):
    p = params
    src = edge_index[0].astype(jnp.int32)
    dst = edge_index[1].astype(jnp.int32)
    loop = jnp.arange(N, dtype=jnp.int32)

    npad_sl = ESL - (320000 + N)
    pad_i_sl = jnp.arange(npad_sl, dtype=jnp.int32)
    src_sl = jnp.concatenate([src, loop, pad_i_sl % 64]).reshape(ESL // 128, 128)
    dst_sl = jnp.concatenate([dst, loop, N + pad_i_sl % (NP - N)]
                             ).reshape(ESL // 128, 128)
    npad_sg = ESG - 320000
    pad_i_sg = jnp.arange(npad_sg, dtype=jnp.int32)
    src_sg = jnp.concatenate([src, pad_i_sg % 64]).reshape(ESG // 128, 128)
    dst_sg = jnp.concatenate([dst, N + pad_i_sg % (NP - N)]
                             ).reshape(ESG // 128, 128)

    xp = jnp.pad(x, ((0, NP - N), (0, 0)))
    zeros = jnp.zeros((NP, H), jnp.float32)
    batchT = jnp.pad(batch.astype(jnp.int32), (0, NP - N),
                     constant_values=-1).reshape(1, NP)

    wcat1 = jnp.concatenate(
        [p['gcn_W1'], p['gat_W1'], p['sage_W1l'], p['sage_W1r']], axis=1)
    a1 = jnp.zeros((H, 128), jnp.float32)
    a1 = a1.at[:, 0].set(p['gat_as1']).at[:, 1].set(p['gat_ad1'])
    a2 = jnp.zeros((H, 128), jnp.float32)
    a2 = a2.at[:, 0].set(p['gat_as2']).at[:, 1].set(p['gat_ad2'])
    wslr2 = jnp.concatenate([p['sage_W2l'], p['sage_W2r']], axis=1)
    b1 = jnp.stack([p['gcn_b1'], p['gat_b1'], p['sage_b1']])
    b2 = jnp.stack([p['gcn_b2'], p['gat_b2'], p['sage_b2']])
    wfc = jnp.zeros((3 * H, 128), jnp.float32)
    wfc = wfc.at[0:H, 0:C].set(p['gcn_fcW'])
    wfc = wfc.at[H:2 * H, 0:C].set(p['gat_fcW'])
    wfc = wfc.at[2 * H:3 * H, 0:C].set(p['sage_fcW'])
    fcb = jnp.zeros((1, 128), jnp.float32)
    fcb = fcb.at[0, 0:C].set(p['gcn_fcb'] + p['gat_fcb'] + p['sage_fcb'])

    degp = _make_deg(ESG)(dst_sg)
    degT = degp.T

    tg1, ta1, ts1, xr1, sd1 = _tc1(xp, degT, wcat1, a1)
    s1 = sd1[:, 0] + 0.0
    d1 = sd1[:, 1] + 0.0

    rowpass_sl = _make_rowpass(ESL)
    rowpass_sg = _make_rowpass(ESG)
    gat_sl = _make_gat(ESL)

    accg1 = rowpass_sl(tg1, src_sl, dst_sl, zeros)
    acca1, den1 = gat_sl(ta1, s1, d1, src_sl, dst_sl, zeros)
    accs1 = rowpass_sg(ts1, src_sg, dst_sg, zeros)

    tg2, ta2, ts2, xr2, sd2 = _tc2(accg1, acca1, accs1, den1.T, degT, xr1,
                                   p['gcn_W2'], p['gat_W2'], wslr2, a2, b1)
    s2 = sd2[:, 0] + 0.0
    d2 = sd2[:, 1] + 0.0

    accg2 = rowpass_sl(tg2, src_sl, dst_sl, zeros)
    acca2, den2 = gat_sl(ta2, s2, d2, src_sl, dst_sl, zeros)
    accs2 = rowpass_sg(ts2, src_sg, dst_sg, zeros)

    out128 = _tc3(accg2, acca2, accs2, den2.T, degT, xr2, batchT, b2, wfc, fcb)
    return out128[:, :C]


# speedup vs baseline: 52.5593x; 1.0245x over previous
import functools

import jax
import jax.numpy as jnp
from jax import lax
from jax.experimental import pallas as pl
from jax.experimental.pallas import tpu as pltpu
from jax.experimental.pallas import tpu_sc as plsc

N = 10000
D = 128
H = 64
C = 2
G = 128

NP = 10240
NC, NS = 2, 16
NW = NC * NS
BLK = 512
ROWS_PER_TILE = NP // NS

_mesh = plsc.VectorSubcoreMesh(core_axis_name="c", subcore_axis_name="s")


def _epad(n_edges):
    per = NW * BLK
    return ((n_edges + per - 1) // per) * per


ESL = _epad(320000 + N)
ESG = _epad(320000)


def _splat(vec, lane):
    return lax.gather(
        vec, jnp.full((16, 1), lane, dtype=jnp.int32),
        lax.GatherDimensionNumbers(offset_dims=(), collapsed_slice_dims=(0,),
                                   start_index_map=(0,)),
        (1,), mode=lax.GatherScatterMode.PROMISE_IN_BOUNDS)


def _deg_body(nblk, dst_hbm, out_hbm, idx0, idx1, idx2, ones_v, zb, acc_sh,
              semi, sems):
    c = lax.axis_index("c")
    s = lax.axis_index("s")
    w = c * NS + s
    idxv = [idx0, idx1, idx2]
    for i in range(8):
        ones_v[pl.ds(i * 16, 16)] = jnp.ones((16,), jnp.float32)
    for i in range(ROWS_PER_TILE // 16):
        zb[pl.ds(i * 16, 16)] = jnp.zeros((16,), jnp.float32)
    pltpu.sync_copy(zb, acc_sh.at[pl.ds(s * ROWS_PER_TILE, ROWS_PER_TILE)])
    plsc.subcore_barrier()

    GPB = BLK // 128
    ngrp = nblk * GPB
    di = {0: pltpu.async_copy(dst_hbm.at[pl.ds(w * nblk * GPB, GPB)],
                              idxv[0], semi)}
    dsc = [None] * ngrp
    for b in range(nblk):
        di[b].wait()
        if b + 1 < nblk:
            r0 = (w * nblk + b + 1) * GPB
            di[b + 1] = pltpu.async_copy(dst_hbm.at[pl.ds(r0, GPB)],
                                         idxv[(b + 1) % 3], semi)
        for j in range(GPB):
            t = GPB * b + j
            if t >= 4:
                dsc[t - 4].wait()
            dsc[t] = pltpu.async_copy(ones_v, acc_sh.at[idxv[b % 3].at[j]],
                                      sems, add=True)
    for t in range(max(ngrp - 4, 0), ngrp):
        dsc[t].wait()
    plsc.subcore_barrier()
    pltpu.sync_copy(acc_sh.at[pl.ds(s * ROWS_PER_TILE, ROWS_PER_TILE)],
                    out_hbm.at[c, pl.ds(s * ROWS_PER_TILE, ROWS_PER_TILE)])


def _make_deg(n_edges_pad):
    nblk = n_edges_pad // (NW * BLK)
    return pl.kernel(
        functools.partial(_deg_body, nblk),
        out_type=jax.ShapeDtypeStruct((NC, NP), jnp.float32),
        mesh=_mesh,
        compiler_params=pltpu.CompilerParams(needs_layout_passes=False, use_tc_tiling_on_sc=False),
        scratch_types=[
            pltpu.VMEM((BLK // 128, 128), jnp.int32),
            pltpu.VMEM((BLK // 128, 128), jnp.int32),
            pltpu.VMEM((BLK // 128, 128), jnp.int32),
            pltpu.VMEM((128,), jnp.float32),
            pltpu.VMEM((ROWS_PER_TILE,), jnp.float32),
            pltpu.VMEM_SHARED((NP,), jnp.float32),
            pltpu.SemaphoreType.DMA,
            pltpu.SemaphoreType.DMA,
        ],
    )


def _rowpass_body(nblk, table_hbm, src_hbm, dst_hbm, zeros_hbm, out_hbm,
                  s0, s1, s2, s3, d0, d1, d2, d3,
                  r0_, r1_, r2_, r3_, r4_, r5_, r6_, r7_, acc_sh,
                  semi, semg, sems):
    c = lax.axis_index("c")
    s = lax.axis_index("s")
    w = c * NS + s
    srcv = [s0, s1, s2, s3]
    dstv = [d0, d1, d2, d3]
    rows = [r0_, r1_, r2_, r3_, r4_, r5_, r6_, r7_]
    r = pl.ds(s * ROWS_PER_TILE, ROWS_PER_TILE)
    pltpu.sync_copy(zeros_hbm.at[r], acc_sh.at[r])
    plsc.subcore_barrier()

    GPB = BLK // 128
    ngrp = nblk * GPB
    base = w * nblk * GPB
    di = {0: (pltpu.async_copy(src_hbm.at[pl.ds(base, GPB)], srcv[0], semi),
              pltpu.async_copy(dst_hbm.at[pl.ds(base, GPB)], dstv[0], semi))}
    dg = [None] * ngrp
    dsc = [None] * ngrp

    def issue_scatter(u):
        dg[u].wait()
        dsc[u] = pltpu.async_copy(rows[u % 8],
                                  acc_sh.at[dstv[(u // GPB) % 4].at[u % GPB]],
                                  sems, add=True)

    for b in range(nblk):
        for dd in di[b]:
            dd.wait()
        if b + 1 < nblk:
            rr = base + (b + 1) * GPB
            di[b + 1] = (pltpu.async_copy(src_hbm.at[pl.ds(rr, GPB)],
                                          srcv[(b + 1) % 4], semi),
                         pltpu.async_copy(dst_hbm.at[pl.ds(rr, GPB)],
                                          dstv[(b + 1) % 4], semi))
        for j in range(GPB):
            t = GPB * b + j
            if t >= 8:
                dsc[t - 8].wait()
            dg[t] = pltpu.async_copy(table_hbm.at[srcv[b % 4].at[j]],
                                     rows[t % 8], semg)
            if t >= 4:
                issue_scatter(t - 4)
    for u in range(max(ngrp - 4, 0), ngrp):
        issue_scatter(u)
    for u in range(max(ngrp - 8, 0), ngrp):
        dsc[u].wait()
    plsc.subcore_barrier()
    pltpu.sync_copy(acc_sh.at[r], out_hbm.at[c, r])


def _make_rowpass(n_edges_pad):
    nblk = n_edges_pad // (NW * BLK)
    return pl.kernel(
        functools.partial(_rowpass_body, nblk),
        out_type=jax.ShapeDtypeStruct((NC, NP, H), jnp.float32),
        mesh=_mesh,
        compiler_params=pltpu.CompilerParams(needs_layout_passes=False, use_tc_tiling_on_sc=False),
        scratch_types=(
            [pltpu.VMEM((BLK // 128, 128), jnp.int32)] * 8
            + [pltpu.VMEM((128, H), jnp.float32)] * 8
            + [pltpu.VMEM_SHARED((NP, H), jnp.float32),
               pltpu.SemaphoreType.DMA,
               pltpu.SemaphoreType.DMA,
               pltpu.SemaphoreType.DMA]
        ),
    )


def _gat_body(nblk, table_hbm, s_hbm, d_hbm, src_hbm, dst_hbm, zeros_hbm,
              out_hbm, den_hbm, src_v, dst_v, r0_, r1_, r2_, r3_, s_t, d_t,
              ex_vs, zb, acc_sh, den_sh, semg, sems):
    c = lax.axis_index("c")
    s = lax.axis_index("s")
    w = c * NS + s
    rows = [r0_, r1_, r2_, r3_]
    r = pl.ds(s * ROWS_PER_TILE, ROWS_PER_TILE)
    pltpu.sync_copy(zeros_hbm.at[r], acc_sh.at[r])
    for i in range(ROWS_PER_TILE // 16):
        zb[pl.ds(i * 16, 16)] = jnp.zeros((16,), jnp.float32)
    pltpu.sync_copy(zb, den_sh.at[r])
    pltpu.sync_copy(s_hbm, s_t)
    pltpu.sync_copy(d_hbm, d_t)
    plsc.subcore_barrier()

    GPB = BLK // 128

    def blk(b, carry):
        rr = (w * nblk + b) * GPB
        pltpu.sync_copy(src_hbm.at[pl.ds(rr, GPB)], src_v)
        pltpu.sync_copy(dst_hbm.at[pl.ds(rr, GPB)], dst_v)
        dg = [pltpu.async_copy(table_hbm.at[src_v.at[j]], rows[j], semg)
              for j in range(GPB)]
        ds = []
        for j in range(GPB):
            dg[j].wait()
            rbuf = rows[j]

            @plsc.parallel_loop(0, 8, unroll=4)
            def _grp(g, j=j, rbuf=rbuf):
                sl = pl.ds(g * 16, 16)
                sv = plsc.load_gather(s_t, [src_v[j, sl]])
                dv = plsc.load_gather(d_t, [dst_v[j, sl]])
                e = sv + dv
                e = jnp.where(e > 0, e, e * 0.2)
                ex = jnp.exp(e)
                ex_vs[j, sl] = ex
                for l in range(16):
                    wv = _splat(ex, l)
                    i = g * 16 + l
                    for q in range(H // 16):
                        qs = pl.ds(q * 16, 16)
                        rbuf[i, qs] = rbuf[i, qs] * wv

            ds.append(pltpu.async_copy(ex_vs.at[j], den_sh.at[dst_v.at[j]],
                                       sems, add=True))
            ds.append(pltpu.async_copy(rbuf, acc_sh.at[dst_v.at[j]],
                                       sems, add=True))
        for d in ds:
            d.wait()
        return carry

    lax.fori_loop(0, nblk, blk, 0)
    plsc.subcore_barrier()
    pltpu.sync_copy(acc_sh.at[r], out_hbm.at[c, r])
    pltpu.sync_copy(den_sh.at[r], den_hbm.at[c, r])


def _make_gat(n_edges_pad):
    nblk = n_edges_pad // (NW * BLK)
    return pl.kernel(
        functools.partial(_gat_body, nblk),
        out_type=(jax.ShapeDtypeStruct((NC, NP, H), jnp.float32),
                  jax.ShapeDtypeStruct((NC, NP), jnp.float32)),
        mesh=_mesh,
        compiler_params=pltpu.CompilerParams(needs_layout_passes=False, use_tc_tiling_on_sc=False),
        scratch_types=(
            [pltpu.VMEM((BLK // 128, 128), jnp.int32)] * 2
            + [pltpu.VMEM((128, H), jnp.float32)] * 4
            + [pltpu.VMEM((NP,), jnp.float32),
               pltpu.VMEM((NP,), jnp.float32),
               pltpu.VMEM((BLK // 128, 128), jnp.float32),
               pltpu.VMEM((ROWS_PER_TILE,), jnp.float32),
               pltpu.VMEM_SHARED((NP, H), jnp.float32),
               pltpu.VMEM_SHARED((NP,), jnp.float32),
               pltpu.SemaphoreType.DMA,
               pltpu.SemaphoreType.DMA]
        ),
    )


RB = 1280
NG = NP // RB


def _tc1_body(xp_ref, degT_ref, wcat_ref, a1_ref,
              tg_ref, ta_ref, ts_ref, xr_ref, sd_ref):
    xb = xp_ref[...]
    h4 = jnp.dot(xb, wcat_ref[...], preferred_element_type=jnp.float32)
    deg = degT_ref[:, 0:1] + degT_ref[:, 1:2]
    dinv = lax.rsqrt(deg + 1.0)
    tg_ref[...] = h4[:, 0:H] * dinv
    ta = h4[:, H:2 * H]
    ta_ref[...] = ta
    ts_ref[...] = h4[:, 2 * H:3 * H]
    xr_ref[...] = h4[:, 3 * H:4 * H]
    sd_ref[...] = jnp.dot(ta, a1_ref[...], preferred_element_type=jnp.float32)


def _tc1(xp, degT, wcat, a1):
    f = pl.pallas_call(
        _tc1_body,
        grid=(NG,),
        in_specs=[
            pl.BlockSpec((RB, D), lambda i: (i, 0)),
            pl.BlockSpec((RB, 2), lambda i: (i, 0)),
            pl.BlockSpec((D, 4 * H), lambda i: (0, 0)),
            pl.BlockSpec((H, 128), lambda i: (0, 0)),
        ],
        out_specs=[
            pl.BlockSpec((RB, H), lambda i: (i, 0)),
            pl.BlockSpec((RB, H), lambda i: (i, 0)),
            pl.BlockSpec((RB, H), lambda i: (i, 0)),
            pl.BlockSpec((RB, H), lambda i: (i, 0)),
            pl.BlockSpec((RB, 128), lambda i: (i, 0)),
        ],
        out_shape=[jax.ShapeDtypeStruct((NP, H), jnp.float32)] * 4
        + [jax.ShapeDtypeStruct((NP, 128), jnp.float32)],
    )
    return f(xp, degT, wcat, a1)


def _tc2_body(accg_ref, acca_ref, accs_ref, denT_ref, degT_ref, xr1_ref,
              wg_ref, wa_ref, wslr_ref, a2_ref, b1_ref,
              tg2_ref, ta2_ref, ts2_ref, xr2_ref, sd2_ref):
    deg = degT_ref[:, 0:1] + degT_ref[:, 1:2]
    dinv = lax.rsqrt(deg + 1.0)
    den = denT_ref[:, 0:1] + denT_ref[:, 1:2]
    h1g = jnp.maximum(dinv * (accg_ref[0] + accg_ref[1]) + b1_ref[0:1, :], 0.0)
    h1a = jnp.maximum((acca_ref[0] + acca_ref[1]) / den + b1_ref[1:2, :], 0.0)
    h1s = jnp.maximum((accs_ref[0] + accs_ref[1]) / jnp.maximum(deg, 1.0)
                      + xr1_ref[...] + b1_ref[2:3, :], 0.0)
    tg2_ref[...] = dinv * jnp.dot(h1g, wg_ref[...],
                                  preferred_element_type=jnp.float32)
    ta2 = jnp.dot(h1a, wa_ref[...], preferred_element_type=jnp.float32)
    ta2_ref[...] = ta2
    hs2 = jnp.dot(h1s, wslr_ref[...], preferred_element_type=jnp.float32)
    ts2_ref[...] = hs2[:, 0:H]
    xr2_ref[...] = hs2[:, H:2 * H]
    sd2_ref[...] = jnp.dot(ta2, a2_ref[...], preferred_element_type=jnp.float32)


def _tc2(accg, acca, accs, denT, degT, xr1, wg, wa, wslr, a2, b1):
    f = pl.pallas_call(
        _tc2_body,
        grid=(NG,),
        in_specs=[
            pl.BlockSpec((NC, RB, H), lambda i: (0, i, 0)),
            pl.BlockSpec((NC, RB, H), lambda i: (0, i, 0)),
            pl.BlockSpec((NC, RB, H), lambda i: (0, i, 0)),
            pl.BlockSpec((RB, 2), lambda i: (i, 0)),
            pl.BlockSpec((RB, 2), lambda i: (i, 0)),
            pl.BlockSpec((RB, H), lambda i: (i, 0)),
            pl.BlockSpec((H, H), lambda i: (0, 0)),
            pl.BlockSpec((H, H), lambda i: (0, 0)),
            pl.BlockSpec((H, 2 * H), lambda i: (0, 0)),
            pl.BlockSpec((H, 128), lambda i: (0, 0)),
            pl.BlockSpec((3, H), lambda i: (0, 0)),
        ],
        out_specs=[
            pl.BlockSpec((RB, H), lambda i: (i, 0)),
            pl.BlockSpec((RB, H), lambda i: (i, 0)),
            pl.BlockSpec((RB, H), lambda i: (i, 0)),
            pl.BlockSpec((RB, H), lambda i: (i, 0)),
            pl.BlockSpec((RB, 128), lambda i: (i, 0)),
        ],
        out_shape=[jax.ShapeDtypeStruct((NP, H), jnp.float32)] * 4
        + [jax.ShapeDtypeStruct((NP, 128), jnp.float32)],
    )
    return f(accg, acca, accs, denT, degT, xr1, wg, wa, wslr, a2, b1)


def _tc3_body(accg_ref, acca_ref, accs_ref, denT_ref, degT_ref, xr2_ref,
              batchT_ref, b2_ref, wfc_ref, fcb_ref, out_ref,
              pooled_ref, cnt_ref):
    i = pl.program_id(0)

    @pl.when(i == 0)
    def _():
        pooled_ref[...] = jnp.zeros_like(pooled_ref)
        cnt_ref[...] = jnp.zeros_like(cnt_ref)

    deg = degT_ref[:, 0:1] + degT_ref[:, 1:2]
    dinv = lax.rsqrt(deg + 1.0)
    den = denT_ref[:, 0:1] + denT_ref[:, 1:2]
    h2g = jnp.maximum(dinv * (accg_ref[0] + accg_ref[1]) + b2_ref[0:1, :], 0.0)
    h2a = jnp.maximum((acca_ref[0] + acca_ref[1]) / den + b2_ref[1:2, :], 0.0)
    h2s = jnp.maximum((accs_ref[0] + accs_ref[1]) / jnp.maximum(deg, 1.0)
                      + xr2_ref[...] + b2_ref[2:3, :], 0.0)
    hcat = jnp.concatenate([h2g, h2a, h2s], axis=1)
    pb = (batchT_ref[...] ==
          lax.broadcasted_iota(jnp.int32, (G, RB), 0)).astype(jnp.float32)
    pooled_ref[...] += jnp.dot(pb, hcat, preferred_element_type=jnp.float32)
    cnt_ref[...] += jnp.sum(pb, axis=1, keepdims=True)

    @pl.when(i == NG - 1)
    def _():
        cnt = jnp.maximum(cnt_ref[...], 1.0)
        pool = pooled_ref[...] / cnt
        out_ref[...] = (jnp.dot(pool, wfc_ref[...],
                                preferred_element_type=jnp.float32)
                        + fcb_ref[...]) * (1.0 / 3.0)


def _tc3(accg, acca, accs, denT, degT, xr2, batchT, b2, wfc, fcb):
    f = pl.pallas_call(
        _tc3_body,
        grid=(NG,),
        in_specs=[
            pl.BlockSpec((NC, RB, H), lambda i: (0, i, 0)),
            pl.BlockSpec((NC, RB, H), lambda i: (0, i, 0)),
            pl.BlockSpec((NC, RB, H), lambda i: (0, i, 0)),
            pl.BlockSpec((RB, 2), lambda i: (i, 0)),
            pl.BlockSpec((RB, 2), lambda i: (i, 0)),
            pl.BlockSpec((RB, H), lambda i: (i, 0)),
            pl.BlockSpec((1, RB), lambda i: (0, i)),
            pl.BlockSpec((3, H), lambda i: (0, 0)),
            pl.BlockSpec((3 * H, 128), lambda i: (0, 0)),
            pl.BlockSpec((1, 128), lambda i: (0, 0)),
        ],
        out_specs=pl.BlockSpec((G, 128), lambda i: (0, 0)),
        out_shape=jax.ShapeDtypeStruct((G, 128), jnp.float32),
        scratch_shapes=[
            pltpu.VMEM((G, 3 * H), jnp.float32),
            pltpu.VMEM((G, 1), jnp.float32),
        ],
    )
    return f(accg, acca, accs, denT, degT, xr2, batchT, b2, wfc, fcb)


def kernel(x, edge_index, batch, params):
    p = params
    src = edge_index[0].astype(jnp.int32)
    dst = edge_index[1].astype(jnp.int32)
    loop = jnp.arange(N, dtype=jnp.int32)

    npad_sl = ESL - (320000 + N)
    pad_i_sl = jnp.arange(npad_sl, dtype=jnp.int32)
    src_sl = jnp.concatenate([src, loop, pad_i_sl % 64]).reshape(ESL // 128, 128)
    dst_sl = jnp.concatenate([dst, loop, N + pad_i_sl % (NP - N)]
                             ).reshape(ESL // 128, 128)
    npad_sg = ESG - 320000
    pad_i_sg = jnp.arange(npad_sg, dtype=jnp.int32)
    src_sg = jnp.concatenate([src, pad_i_sg % 64]).reshape(ESG // 128, 128)
    dst_sg = jnp.concatenate([dst, N + pad_i_sg % (NP - N)]
                             ).reshape(ESG // 128, 128)

    xp = jnp.pad(x, ((0, NP - N), (0, 0)))
    zeros = jnp.zeros((NP, H), jnp.float32)
    batchT = jnp.pad(batch.astype(jnp.int32), (0, NP - N),
                     constant_values=-1).reshape(1, NP)

    wcat1 = jnp.concatenate(
        [p['gcn_W1'], p['gat_W1'], p['sage_W1l'], p['sage_W1r']], axis=1)
    a1 = jnp.zeros((H, 128), jnp.float32)
    a1 = a1.at[:, 0].set(p['gat_as1']).at[:, 1].set(p['gat_ad1'])
    a2 = jnp.zeros((H, 128), jnp.float32)
    a2 = a2.at[:, 0].set(p['gat_as2']).at[:, 1].set(p['gat_ad2'])
    wslr2 = jnp.concatenate([p['sage_W2l'], p['sage_W2r']], axis=1)
    b1 = jnp.stack([p['gcn_b1'], p['gat_b1'], p['sage_b1']])
    b2 = jnp.stack([p['gcn_b2'], p['gat_b2'], p['sage_b2']])
    wfc = jnp.zeros((3 * H, 128), jnp.float32)
    wfc = wfc.at[0:H, 0:C].set(p['gcn_fcW'])
    wfc = wfc.at[H:2 * H, 0:C].set(p['gat_fcW'])
    wfc = wfc.at[2 * H:3 * H, 0:C].set(p['sage_fcW'])
    fcb = jnp.zeros((1, 128), jnp.float32)
    fcb = fcb.at[0, 0:C].set(p['gcn_fcb'] + p['gat_fcb'] + p['sage_fcb'])

    degp = _make_deg(ESG)(dst_sg)
    degT = degp.T

    tg1, ta1, ts1, xr1, sd1 = _tc1(xp, degT, wcat1, a1)
    s1 = sd1[:, 0] + 0.0
    d1 = sd1[:, 1] + 0.0

    rowpass_sl = _make_rowpass(ESL)
    rowpass_sg = _make_rowpass(ESG)
    gat_sl = _make_gat(ESL)

    accg1 = rowpass_sl(tg1, src_sl, dst_sl, zeros)
    acca1, den1 = gat_sl(ta1, s1, d1, src_sl, dst_sl, zeros)
    accs1 = rowpass_sg(ts1, src_sg, dst_sg, zeros)

    tg2, ta2, ts2, xr2, sd2 = _tc2(accg1, acca1, accs1, den1.T, degT, xr1,
                                   p['gcn_W2'], p['gat_W2'], wslr2, a2, b1)
    s2 = sd2[:, 0] + 0.0
    d2 = sd2[:, 1] + 0.0

    accg2 = rowpass_sl(tg2, src_sl, dst_sl, zeros)
    acca2, den2 = gat_sl(ta2, s2, d2, src_sl, dst_sl, zeros)
    accs2 = rowpass_sg(ts2, src_sg, dst_sg, zeros)

    out128 = _tc3(accg2, acca2, accs2, den2.T, degT, xr2, batchT, b2, wfc, fcb)
    return out128[:, :C]


# speedup vs baseline: 54.2484x; 1.0321x over previous
import functools

import jax
import jax.numpy as jnp
from jax import lax
from jax.experimental import pallas as pl
from jax.experimental.pallas import tpu as pltpu
from jax.experimental.pallas import tpu_sc as plsc

N = 10000
D = 128
H = 64
C = 2
G = 128

NP = 10240
NC, NS = 2, 16
NW = NC * NS
BLK = 512
ROWS_PER_TILE = NP // NS

_mesh = plsc.VectorSubcoreMesh(core_axis_name="c", subcore_axis_name="s")


def _epad(n_edges):
    per = NW * BLK
    return ((n_edges + per - 1) // per) * per


ESL = _epad(320000 + N)
ESG = _epad(320000)


def _splat(vec, lane):
    return lax.gather(
        vec, jnp.full((16, 1), lane, dtype=jnp.int32),
        lax.GatherDimensionNumbers(offset_dims=(), collapsed_slice_dims=(0,),
                                   start_index_map=(0,)),
        (1,), mode=lax.GatherScatterMode.PROMISE_IN_BOUNDS)


def _deg_body(nblk, dst_hbm, out_hbm, idx0, idx1, idx2, ones_v, zb, acc_sh,
              semi, sems):
    c = lax.axis_index("c")
    s = lax.axis_index("s")
    w = c * NS + s
    idxv = [idx0, idx1, idx2]
    for i in range(8):
        ones_v[pl.ds(i * 16, 16)] = jnp.ones((16,), jnp.float32)
    for i in range(ROWS_PER_TILE // 16):
        zb[pl.ds(i * 16, 16)] = jnp.zeros((16,), jnp.float32)
    pltpu.sync_copy(zb, acc_sh.at[pl.ds(s * ROWS_PER_TILE, ROWS_PER_TILE)])
    plsc.subcore_barrier()

    GPB = BLK // 128
    ngrp = nblk * GPB
    di = {0: pltpu.async_copy(dst_hbm.at[pl.ds(w * nblk * GPB, GPB)],
                              idxv[0], semi)}
    dsc = [None] * ngrp
    for b in range(nblk):
        di[b].wait()
        if b + 1 < nblk:
            r0 = (w * nblk + b + 1) * GPB
            di[b + 1] = pltpu.async_copy(dst_hbm.at[pl.ds(r0, GPB)],
                                         idxv[(b + 1) % 3], semi)
        for j in range(GPB):
            t = GPB * b + j
            if t >= 4:
                dsc[t - 4].wait()
            dsc[t] = pltpu.async_copy(ones_v, acc_sh.at[idxv[b % 3].at[j]],
                                      sems, add=True)
    for t in range(max(ngrp - 4, 0), ngrp):
        dsc[t].wait()
    plsc.subcore_barrier()
    pltpu.sync_copy(acc_sh.at[pl.ds(s * ROWS_PER_TILE, ROWS_PER_TILE)],
                    out_hbm.at[c, pl.ds(s * ROWS_PER_TILE, ROWS_PER_TILE)])


def _make_deg(n_edges_pad):
    nblk = n_edges_pad // (NW * BLK)
    return pl.kernel(
        functools.partial(_deg_body, nblk),
        out_type=jax.ShapeDtypeStruct((NC, NP), jnp.float32),
        mesh=_mesh,
        compiler_params=pltpu.CompilerParams(needs_layout_passes=False, use_tc_tiling_on_sc=False),
        scratch_types=[
            pltpu.VMEM((BLK // 128, 128), jnp.int32),
            pltpu.VMEM((BLK // 128, 128), jnp.int32),
            pltpu.VMEM((BLK // 128, 128), jnp.int32),
            pltpu.VMEM((128,), jnp.float32),
            pltpu.VMEM((ROWS_PER_TILE,), jnp.float32),
            pltpu.VMEM_SHARED((NP,), jnp.float32),
            pltpu.SemaphoreType.DMA,
            pltpu.SemaphoreType.DMA,
        ],
    )


def _rowpass_body(nblk, with_deg, table_hbm, src_hbm, dst_hbm, zeros_hbm,
                  *refs):
    if with_deg:
        (out_hbm, deg_hbm, s0, s1, s2, s3, d0, d1, d2, d3,
         r0_, r1_, r2_, r3_, r4_, r5_, r6_, r7_, ones_v, zb, acc_sh, deg_sh,
         semi, semg, sems) = refs
    else:
        (out_hbm, s0, s1, s2, s3, d0, d1, d2, d3,
         r0_, r1_, r2_, r3_, r4_, r5_, r6_, r7_, acc_sh,
         semi, semg, sems) = refs
    c = lax.axis_index("c")
    s = lax.axis_index("s")
    w = c * NS + s
    srcv = [s0, s1, s2, s3]
    dstv = [d0, d1, d2, d3]
    rows = [r0_, r1_, r2_, r3_, r4_, r5_, r6_, r7_]
    r = pl.ds(s * ROWS_PER_TILE, ROWS_PER_TILE)
    pltpu.sync_copy(zeros_hbm.at[r], acc_sh.at[r])
    if with_deg:
        for i in range(8):
            ones_v[pl.ds(i * 16, 16)] = jnp.ones((16,), jnp.float32)
        for i in range(ROWS_PER_TILE // 16):
            zb[pl.ds(i * 16, 16)] = jnp.zeros((16,), jnp.float32)
        pltpu.sync_copy(zb, deg_sh.at[r])
    plsc.subcore_barrier()

    GPB = BLK // 128
    ngrp = nblk * GPB
    base = w * nblk * GPB
    di = {0: (pltpu.async_copy(src_hbm.at[pl.ds(base, GPB)], srcv[0], semi),
              pltpu.async_copy(dst_hbm.at[pl.ds(base, GPB)], dstv[0], semi))}
    dg = [None] * ngrp
    dsc = [None] * ngrp

    dsd = [None] * ngrp

    def issue_scatter(u):
        dg[u].wait()
        dsc[u] = pltpu.async_copy(rows[u % 8],
                                  acc_sh.at[dstv[(u // GPB) % 4].at[u % GPB]],
                                  sems, add=True)
        if with_deg:
            dsd[u] = pltpu.async_copy(ones_v,
                                      deg_sh.at[dstv[(u // GPB) % 4].at[u % GPB]],
                                      sems, add=True)

    for b in range(nblk):
        for dd in di[b]:
            dd.wait()
        if b + 1 < nblk:
            rr = base + (b + 1) * GPB
            di[b + 1] = (pltpu.async_copy(src_hbm.at[pl.ds(rr, GPB)],
                                          srcv[(b + 1) % 4], semi),
                         pltpu.async_copy(dst_hbm.at[pl.ds(rr, GPB)],
                                          dstv[(b + 1) % 4], semi))
        for j in range(GPB):
            t = GPB * b + j
            if t >= 8:
                dsc[t - 8].wait()
                if with_deg:
                    dsd[t - 8].wait()
            dg[t] = pltpu.async_copy(table_hbm.at[srcv[b % 4].at[j]],
                                     rows[t % 8], semg)
            if t >= 4:
                issue_scatter(t - 4)
    for u in range(max(ngrp - 4, 0), ngrp):
        issue_scatter(u)
    for u in range(max(ngrp - 8, 0), ngrp):
        dsc[u].wait()
        if with_deg:
            dsd[u].wait()
    plsc.subcore_barrier()
    pltpu.sync_copy(acc_sh.at[r], out_hbm.at[c, r])
    if with_deg:
        pltpu.sync_copy(deg_sh.at[r], deg_hbm.at[c, r])


def _make_rowpass(n_edges_pad, with_deg=False):
    nblk = n_edges_pad // (NW * BLK)
    out_type = jax.ShapeDtypeStruct((NC, NP, H), jnp.float32)
    if with_deg:
        out_type = (out_type, jax.ShapeDtypeStruct((NC, NP), jnp.float32))
    extra = ([pltpu.VMEM((128,), jnp.float32),
              pltpu.VMEM((ROWS_PER_TILE,), jnp.float32)] if with_deg else [])
    extra_sh = ([pltpu.VMEM_SHARED((NP,), jnp.float32)] if with_deg else [])
    return pl.kernel(
        functools.partial(_rowpass_body, nblk, with_deg),
        out_type=out_type,
        mesh=_mesh,
        compiler_params=pltpu.CompilerParams(needs_layout_passes=False, use_tc_tiling_on_sc=False),
        scratch_types=(
            [pltpu.VMEM((BLK // 128, 128), jnp.int32)] * 8
            + [pltpu.VMEM((128, H), jnp.float32)] * 8
            + extra
            + [pltpu.VMEM_SHARED((NP, H), jnp.float32)]
            + extra_sh
            + [pltpu.SemaphoreType.DMA,
               pltpu.SemaphoreType.DMA,
               pltpu.SemaphoreType.DMA]
        ),
    )


def _gat_body(nblk, table_hbm, s_hbm, d_hbm, src_hbm, dst_hbm, zeros_hbm,
              out_hbm, den_hbm, src_v, dst_v, r0_, r1_, r2_, r3_, s_t, d_t,
              ex_vs, zb, acc_sh, den_sh, semg, sems):
    c = lax.axis_index("c")
    s = lax.axis_index("s")
    w = c * NS + s
    rows = [r0_, r1_, r2_, r3_]
    r = pl.ds(s * ROWS_PER_TILE, ROWS_PER_TILE)
    pltpu.sync_copy(zeros_hbm.at[r], acc_sh.at[r])
    for i in range(ROWS_PER_TILE // 16):
        zb[pl.ds(i * 16, 16)] = jnp.zeros((16,), jnp.float32)
    pltpu.sync_copy(zb, den_sh.at[r])
    pltpu.sync_copy(s_hbm, s_t)
    pltpu.sync_copy(d_hbm, d_t)
    plsc.subcore_barrier()

    GPB = BLK // 128

    def blk(b, carry):
        rr = (w * nblk + b) * GPB
        pltpu.sync_copy(src_hbm.at[pl.ds(rr, GPB)], src_v)
        pltpu.sync_copy(dst_hbm.at[pl.ds(rr, GPB)], dst_v)
        dg = [pltpu.async_copy(table_hbm.at[src_v.at[j]], rows[j], semg)
              for j in range(GPB)]
        ds = []
        for j in range(GPB):
            dg[j].wait()
            rbuf = rows[j]

            @plsc.parallel_loop(0, 8, unroll=4)
            def _grp(g, j=j, rbuf=rbuf):
                sl = pl.ds(g * 16, 16)
                sv = plsc.load_gather(s_t, [src_v[j, sl]])
                dv = plsc.load_gather(d_t, [dst_v[j, sl]])
                e = sv + dv
                e = jnp.where(e > 0, e, e * 0.2)
                ex = jnp.exp(e)
                ex_vs[j, sl] = ex
                for l in range(16):
                    wv = _splat(ex, l)
                    i = g * 16 + l
                    for q in range(H // 16):
                        qs = pl.ds(q * 16, 16)
                        rbuf[i, qs] = rbuf[i, qs] * wv

            ds.append(pltpu.async_copy(ex_vs.at[j], den_sh.at[dst_v.at[j]],
                                       sems, add=True))
            ds.append(pltpu.async_copy(rbuf, acc_sh.at[dst_v.at[j]],
                                       sems, add=True))
        for d in ds:
            d.wait()
        return carry

    lax.fori_loop(0, nblk, blk, 0)
    plsc.subcore_barrier()
    pltpu.sync_copy(acc_sh.at[r], out_hbm.at[c, r])
    pltpu.sync_copy(den_sh.at[r], den_hbm.at[c, r])


def _make_gat(n_edges_pad):
    nblk = n_edges_pad // (NW * BLK)
    return pl.kernel(
        functools.partial(_gat_body, nblk),
        out_type=(jax.ShapeDtypeStruct((NC, NP, H), jnp.float32),
                  jax.ShapeDtypeStruct((NC, NP), jnp.float32)),
        mesh=_mesh,
        compiler_params=pltpu.CompilerParams(needs_layout_passes=False, use_tc_tiling_on_sc=False),
        scratch_types=(
            [pltpu.VMEM((BLK // 128, 128), jnp.int32)] * 2
            + [pltpu.VMEM((128, H), jnp.float32)] * 4
            + [pltpu.VMEM((NP,), jnp.float32),
               pltpu.VMEM((NP,), jnp.float32),
               pltpu.VMEM((BLK // 128, 128), jnp.float32),
               pltpu.VMEM((ROWS_PER_TILE,), jnp.float32),
               pltpu.VMEM_SHARED((NP, H), jnp.float32),
               pltpu.VMEM_SHARED((NP,), jnp.float32),
               pltpu.SemaphoreType.DMA,
               pltpu.SemaphoreType.DMA]
        ),
    )


RB = 1280
NG = NP // RB


def _tc1_body(xp_ref, wcat_ref, a1_ref,
              tg_ref, ta_ref, ts_ref, xr_ref, sd_ref):
    xb = xp_ref[...]
    h4 = jnp.dot(xb, wcat_ref[...], preferred_element_type=jnp.float32)
    tg_ref[...] = h4[:, 0:H]
    ta = h4[:, H:2 * H]
    ta_ref[...] = ta
    ts_ref[...] = h4[:, 2 * H:3 * H]
    xr_ref[...] = h4[:, 3 * H:4 * H]
    sd_ref[...] = jnp.dot(ta, a1_ref[...], preferred_element_type=jnp.float32)


def _dinv_body(t_ref, degT_ref, out_ref):
    deg = degT_ref[:, 0:1] + degT_ref[:, 1:2]
    out_ref[...] = t_ref[...] * lax.rsqrt(deg + 1.0)


def _dinv_scale(t, degT):
    f = pl.pallas_call(
        _dinv_body,
        grid=(NG,),
        in_specs=[
            pl.BlockSpec((RB, H), lambda i: (i, 0)),
            pl.BlockSpec((RB, 2), lambda i: (i, 0)),
        ],
        out_specs=pl.BlockSpec((RB, H), lambda i: (i, 0)),
        out_shape=jax.ShapeDtypeStruct((NP, H), jnp.float32),
    )
    return f(t, degT)


def _tc1(xp, wcat, a1):
    f = pl.pallas_call(
        _tc1_body,
        grid=(NG,),
        in_specs=[
            pl.BlockSpec((RB, D), lambda i: (i, 0)),
            pl.BlockSpec((D, 4 * H), lambda i: (0, 0)),
            pl.BlockSpec((H, 128), lambda i: (0, 0)),
        ],
        out_specs=[
            pl.BlockSpec((RB, H), lambda i: (i, 0)),
            pl.BlockSpec((RB, H), lambda i: (i, 0)),
            pl.BlockSpec((RB, H), lambda i: (i, 0)),
            pl.BlockSpec((RB, H), lambda i: (i, 0)),
            pl.BlockSpec((RB, 128), lambda i: (i, 0)),
        ],
        out_shape=[jax.ShapeDtypeStruct((NP, H), jnp.float32)] * 4
        + [jax.ShapeDtypeStruct((NP, 128), jnp.float32)],
    )
    return f(xp, wcat, a1)


def _tc2_body(accg_ref, acca_ref, accs_ref, denT_ref, degT_ref, xr1_ref,
              wg_ref, wa_ref, wslr_ref, a2_ref, b1_ref,
              tg2_ref, ta2_ref, ts2_ref, xr2_ref, sd2_ref):
    deg = degT_ref[:, 0:1] + degT_ref[:, 1:2]
    dinv = lax.rsqrt(deg + 1.0)
    den = denT_ref[:, 0:1] + denT_ref[:, 1:2]
    h1g = jnp.maximum(dinv * (accg_ref[0] + accg_ref[1]) + b1_ref[0:1, :], 0.0)
    h1a = jnp.maximum((acca_ref[0] + acca_ref[1]) / den + b1_ref[1:2, :], 0.0)
    h1s = jnp.maximum((accs_ref[0] + accs_ref[1]) / jnp.maximum(deg, 1.0)
                      + xr1_ref[...] + b1_ref[2:3, :], 0.0)
    tg2_ref[...] = dinv * jnp.dot(h1g, wg_ref[...],
                                  preferred_element_type=jnp.float32)
    ta2 = jnp.dot(h1a, wa_ref[...], preferred_element_type=jnp.float32)
    ta2_ref[...] = ta2
    hs2 = jnp.dot(h1s, wslr_ref[...], preferred_element_type=jnp.float32)
    ts2_ref[...] = hs2[:, 0:H]
    xr2_ref[...] = hs2[:, H:2 * H]
    sd2_ref[...] = jnp.dot(ta2, a2_ref[...], preferred_element_type=jnp.float32)


def _tc2(accg, acca, accs, denT, degT, xr1, wg, wa, wslr, a2, b1):
    f = pl.pallas_call(
        _tc2_body,
        grid=(NG,),
        in_specs=[
            pl.BlockSpec((NC, RB, H), lambda i: (0, i, 0)),
            pl.BlockSpec((NC, RB, H), lambda i: (0, i, 0)),
            pl.BlockSpec((NC, RB, H), lambda i: (0, i, 0)),
            pl.BlockSpec((RB, 2), lambda i: (i, 0)),
            pl.BlockSpec((RB, 2), lambda i: (i, 0)),
            pl.BlockSpec((RB, H), lambda i: (i, 0)),
            pl.BlockSpec((H, H), lambda i: (0, 0)),
            pl.BlockSpec((H, H), lambda i: (0, 0)),
            pl.BlockSpec((H, 2 * H), lambda i: (0, 0)),
            pl.BlockSpec((H, 128), lambda i: (0, 0)),
            pl.BlockSpec((3, H), lambda i: (0, 0)),
        ],
        out_specs=[
            pl.BlockSpec((RB, H), lambda i: (i, 0)),
            pl.BlockSpec((RB, H), lambda i: (i, 0)),
            pl.BlockSpec((RB, H), lambda i: (i, 0)),
            pl.BlockSpec((RB, H), lambda i: (i, 0)),
            pl.BlockSpec((RB, 128), lambda i: (i, 0)),
        ],
        out_shape=[jax.ShapeDtypeStruct((NP, H), jnp.float32)] * 4
        + [jax.ShapeDtypeStruct((NP, 128), jnp.float32)],
    )
    return f(accg, acca, accs, denT, degT, xr1, wg, wa, wslr, a2, b1)


def _tc3_body(accg_ref, acca_ref, accs_ref, denT_ref, degT_ref, xr2_ref,
              batchT_ref, b2_ref, wfc_ref, fcb_ref, out_ref,
              pooled_ref, cnt_ref):
    i = pl.program_id(0)

    @pl.when(i == 0)
    def _():
        pooled_ref[...] = jnp.zeros_like(pooled_ref)
        cnt_ref[...] = jnp.zeros_like(cnt_ref)

    deg = degT_ref[:, 0:1] + degT_ref[:, 1:2]
    dinv = lax.rsqrt(deg + 1.0)
    den = denT_ref[:, 0:1] + denT_ref[:, 1:2]
    h2g = jnp.maximum(dinv * (accg_ref[0] + accg_ref[1]) + b2_ref[0:1, :], 0.0)
    h2a = jnp.maximum((acca_ref[0] + acca_ref[1]) / den + b2_ref[1:2, :], 0.0)
    h2s = jnp.maximum((accs_ref[0] + accs_ref[1]) / jnp.maximum(deg, 1.0)
                      + xr2_ref[...] + b2_ref[2:3, :], 0.0)
    hcat = jnp.concatenate([h2g, h2a, h2s], axis=1)
    pb = (batchT_ref[...] ==
          lax.broadcasted_iota(jnp.int32, (G, RB), 0)).astype(jnp.float32)
    pooled_ref[...] += jnp.dot(pb, hcat, preferred_element_type=jnp.float32)
    cnt_ref[...] += jnp.sum(pb, axis=1, keepdims=True)

    @pl.when(i == NG - 1)
    def _():
        cnt = jnp.maximum(cnt_ref[...], 1.0)
        pool = pooled_ref[...] / cnt
        out_ref[...] = (jnp.dot(pool, wfc_ref[...],
                                preferred_element_type=jnp.float32)
                        + fcb_ref[...]) * (1.0 / 3.0)


def _tc3(accg, acca, accs, denT, degT, xr2, batchT, b2, wfc, fcb):
    f = pl.pallas_call(
        _tc3_body,
        grid=(NG,),
        in_specs=[
            pl.BlockSpec((NC, RB, H), lambda i: (0, i, 0)),
            pl.BlockSpec((NC, RB, H), lambda i: (0, i, 0)),
            pl.BlockSpec((NC, RB, H), lambda i: (0, i, 0)),
            pl.BlockSpec((RB, 2), lambda i: (i, 0)),
            pl.BlockSpec((RB, 2), lambda i: (i, 0)),
            pl.BlockSpec((RB, H), lambda i: (i, 0)),
            pl.BlockSpec((1, RB), lambda i: (0, i)),
            pl.BlockSpec((3, H), lambda i: (0, 0)),
            pl.BlockSpec((3 * H, 128), lambda i: (0, 0)),
            pl.BlockSpec((1, 128), lambda i: (0, 0)),
        ],
        out_specs=pl.BlockSpec((G, 128), lambda i: (0, 0)),
        out_shape=jax.ShapeDtypeStruct((G, 128), jnp.float32),
        scratch_shapes=[
            pltpu.VMEM((G, 3 * H), jnp.float32),
            pltpu.VMEM((G, 1), jnp.float32),
        ],
    )
    return f(accg, acca, accs, denT, degT, xr2, batchT, b2, wfc, fcb)


def kernel(x, edge_index, batch, params):
    p = params
    src = edge_index[0].astype(jnp.int32)
    dst = edge_index[1].astype(jnp.int32)
    loop = jnp.arange(N, dtype=jnp.int32)

    npad_sl = ESL - (320000 + N)
    pad_i_sl = jnp.arange(npad_sl, dtype=jnp.int32)
    src_sl = jnp.concatenate([src, loop, pad_i_sl % 64]).reshape(ESL // 128, 128)
    dst_sl = jnp.concatenate([dst, loop, N + pad_i_sl % (NP - N)]
                             ).reshape(ESL // 128, 128)
    npad_sg = ESG - 320000
    pad_i_sg = jnp.arange(npad_sg, dtype=jnp.int32)
    src_sg = jnp.concatenate([src, pad_i_sg % 64]).reshape(ESG // 128, 128)
    dst_sg = jnp.concatenate([dst, N + pad_i_sg % (NP - N)]
                             ).reshape(ESG // 128, 128)

    xp = jnp.pad(x, ((0, NP - N), (0, 0)))
    zeros = jnp.zeros((NP, H), jnp.float32)
    batchT = jnp.pad(batch.astype(jnp.int32), (0, NP - N),
                     constant_values=-1).reshape(1, NP)

    wcat1 = jnp.concatenate(
        [p['gcn_W1'], p['gat_W1'], p['sage_W1l'], p['sage_W1r']], axis=1)
    a1 = jnp.zeros((H, 128), jnp.float32)
    a1 = a1.at[:, 0].set(p['gat_as1']).at[:, 1].set(p['gat_ad1'])
    a2 = jnp.zeros((H, 128), jnp.float32)
    a2 = a2.at[:, 0].set(p['gat_as2']).at[:, 1].set(p['gat_ad2'])
    wslr2 = jnp.concatenate([p['sage_W2l'], p['sage_W2r']], axis=1)
    b1 = jnp.stack([p['gcn_b1'], p['gat_b1'], p['sage_b1']])
    b2 = jnp.stack([p['gcn_b2'], p['gat_b2'], p['sage_b2']])
    wfc = jnp.zeros((3 * H, 128), jnp.float32)
    wfc = wfc.at[0:H, 0:C].set(p['gcn_fcW'])
    wfc = wfc.at[H:2 * H, 0:C].set(p['gat_fcW'])
    wfc = wfc.at[2 * H:3 * H, 0:C].set(p['sage_fcW'])
    fcb = jnp.zeros((1, 128), jnp.float32)
    fcb = fcb.at[0, 0:C].set(p['gcn_fcb'] + p['gat_fcb'] + p['sage_fcb'])

    tg1r, ta1, ts1, xr1, sd1 = _tc1(xp, wcat1, a1)
    s1 = sd1[:, 0] + 0.0
    d1 = sd1[:, 1] + 0.0

    rowpass_sl = _make_rowpass(ESL)
    rowpass_sgd = _make_rowpass(ESG, with_deg=True)
    rowpass_sg = _make_rowpass(ESG)
    gat_sl = _make_gat(ESL)

    accs1, degp = rowpass_sgd(ts1, src_sg, dst_sg, zeros)
    degT = degp.T
    acca1, den1 = gat_sl(ta1, s1, d1, src_sl, dst_sl, zeros)
    tg1 = _dinv_scale(tg1r, degT)
    accg1 = rowpass_sl(tg1, src_sl, dst_sl, zeros)

    tg2, ta2, ts2, xr2, sd2 = _tc2(accg1, acca1, accs1, den1.T, degT, xr1,
                                   p['gcn_W2'], p['gat_W2'], wslr2, a2, b1)
    s2 = sd2[:, 0] + 0.0
    d2 = sd2[:, 1] + 0.0

    accg2 = rowpass_sl(tg2, src_sl, dst_sl, zeros)
    acca2, den2 = gat_sl(ta2, s2, d2, src_sl, dst_sl, zeros)
    accs2 = rowpass_sg(ts2, src_sg, dst_sg, zeros)

    out128 = _tc3(accg2, acca2, accs2, den2.T, degT, xr2, batchT, b2, wfc, fcb)
    return out128[:, :C]


# speedup vs baseline: 54.3299x; 1.0015x over previous
import functools

import jax
import jax.numpy as jnp
from jax import lax
from jax.experimental import pallas as pl
from jax.experimental.pallas import tpu as pltpu
from jax.experimental.pallas import tpu_sc as plsc

N = 10000
D = 128
H = 64
C = 2
G = 128

NP = 10240
NC, NS = 2, 16
NW = NC * NS
BLK = 512
ROWS_PER_TILE = NP // NS

_mesh = plsc.VectorSubcoreMesh(core_axis_name="c", subcore_axis_name="s")


def _epad(n_edges):
    per = NW * BLK
    return ((n_edges + per - 1) // per) * per


ESL = _epad(320000 + N)
ESG = _epad(320000)


def _splat(vec, lane):
    return lax.gather(
        vec, jnp.full((16, 1), lane, dtype=jnp.int32),
        lax.GatherDimensionNumbers(offset_dims=(), collapsed_slice_dims=(0,),
                                   start_index_map=(0,)),
        (1,), mode=lax.GatherScatterMode.PROMISE_IN_BOUNDS)


NBLK_SL = ESL // (NW * BLK)
NBLK_SG = ESG // (NW * BLK)
GPB = BLK // 128


def _stream_phase(w, nblk, with_deg, table_hbm, src_hbm, dst_hbm,
                  acc_sh, deg_sh, ones_v, srcv, dstv, rows, semi, semg, sems):
    ngrp = nblk * GPB
    base = w * nblk * GPB
    di = {0: (pltpu.async_copy(src_hbm.at[pl.ds(base, GPB)], srcv[0], semi),
              pltpu.async_copy(dst_hbm.at[pl.ds(base, GPB)], dstv[0], semi))}
    dg = [None] * ngrp
    dsc = [None] * ngrp
    dsd = [None] * ngrp

    def issue_scatter(u):
        dg[u].wait()
        dsc[u] = pltpu.async_copy(rows[u % 8],
                                  acc_sh.at[dstv[(u // GPB) % 4].at[u % GPB]],
                                  sems, add=True)
        if with_deg:
            dsd[u] = pltpu.async_copy(
                ones_v, deg_sh.at[dstv[(u // GPB) % 4].at[u % GPB]],
                sems, add=True)

    for b in range(nblk):
        for dd in di[b]:
            dd.wait()
        if b + 1 < nblk:
            rr = base + (b + 1) * GPB
            di[b + 1] = (pltpu.async_copy(src_hbm.at[pl.ds(rr, GPB)],
                                          srcv[(b + 1) % 4], semi),
                         pltpu.async_copy(dst_hbm.at[pl.ds(rr, GPB)],
                                          dstv[(b + 1) % 4], semi))
        for j in range(GPB):
            t = GPB * b + j
            if t >= 8:
                dsc[t - 8].wait()
                if with_deg:
                    dsd[t - 8].wait()
            dg[t] = pltpu.async_copy(table_hbm.at[srcv[b % 4].at[j]],
                                     rows[t % 8], semg)
            if t >= 4:
                issue_scatter(t - 4)
    for u in range(max(ngrp - 4, 0), ngrp):
        issue_scatter(u)
    for u in range(max(ngrp - 8, 0), ngrp):
        dsc[u].wait()
        if with_deg:
            dsd[u].wait()


def _gat_phase(w, nblk, table_hbm, src_hbm, dst_hbm, acc_sh, den_sh,
               src_v, dst_v, rows, s_t, d_t, ex_vs, semg, sems):
    def blk(b, carry):
        rr = (w * nblk + b) * GPB
        pltpu.sync_copy(src_hbm.at[pl.ds(rr, GPB)], src_v)
        pltpu.sync_copy(dst_hbm.at[pl.ds(rr, GPB)], dst_v)
        dg = [pltpu.async_copy(table_hbm.at[src_v.at[j]], rows[j], semg)
              for j in range(GPB)]
        ds = []
        for j in range(GPB):
            dg[j].wait()
            rbuf = rows[j]

            @plsc.parallel_loop(0, 8, unroll=4)
            def _grp(g, j=j, rbuf=rbuf):
                sl = pl.ds(g * 16, 16)
                sv = plsc.load_gather(s_t, [src_v[j, sl]])
                dv = plsc.load_gather(d_t, [dst_v[j, sl]])
                e = sv + dv
                e = jnp.where(e > 0, e, e * 0.2)
                ex = jnp.exp(e)
                ex_vs[j, sl] = ex
                for l in range(16):
                    wv = _splat(ex, l)
                    i = g * 16 + l
                    for q in range(H // 16):
                        qs = pl.ds(q * 16, 16)
                        rbuf[i, qs] = rbuf[i, qs] * wv

            ds.append(pltpu.async_copy(ex_vs.at[j], den_sh.at[dst_v.at[j]],
                                       sems, add=True))
            ds.append(pltpu.async_copy(rbuf, acc_sh.at[dst_v.at[j]],
                                       sems, add=True))
        for d in ds:
            d.wait()
        return carry

    lax.fori_loop(0, nblk, blk, 0)


def _sagedeg_body(ts_hbm, src_hbm, dst_hbm, zeros_hbm, outs_hbm, deg_hbm,
                  s0, s1, s2, s3, d0, d1, d2, d3,
                  r0_, r1_, r2_, r3_, r4_, r5_, r6_, r7_,
                  ones_v, zb, acc_sh, deg_sh, semi, semg, sems):
    c = lax.axis_index("c")
    s = lax.axis_index("s")
    w = c * NS + s
    srcv = [s0, s1, s2, s3]
    dstv = [d0, d1, d2, d3]
    rows = [r0_, r1_, r2_, r3_, r4_, r5_, r6_, r7_]
    r = pl.ds(s * ROWS_PER_TILE, ROWS_PER_TILE)
    for i in range(8):
        ones_v[pl.ds(i * 16, 16)] = jnp.ones((16,), jnp.float32)
    for i in range(ROWS_PER_TILE // 16):
        zb[pl.ds(i * 16, 16)] = jnp.zeros((16,), jnp.float32)
    pltpu.sync_copy(zeros_hbm.at[r], acc_sh.at[r])
    pltpu.sync_copy(zb, deg_sh.at[r])
    plsc.subcore_barrier()
    _stream_phase(w, NBLK_SG, True, ts_hbm, src_hbm, dst_hbm,
                  acc_sh, deg_sh, ones_v, srcv, dstv, rows, semi, semg, sems)
    plsc.subcore_barrier()
    pltpu.sync_copy(acc_sh.at[r], outs_hbm.at[c, r])
    pltpu.sync_copy(deg_sh.at[r], deg_hbm.at[c, r])


def _make_sagedeg():
    return pl.kernel(
        _sagedeg_body,
        out_type=(jax.ShapeDtypeStruct((NC, NP, H), jnp.float32),
                  jax.ShapeDtypeStruct((NC, NP), jnp.float32)),
        mesh=_mesh,
        compiler_params=pltpu.CompilerParams(needs_layout_passes=False, use_tc_tiling_on_sc=False),
        scratch_types=(
            [pltpu.VMEM((GPB, 128), jnp.int32)] * 8
            + [pltpu.VMEM((128, H), jnp.float32)] * 8
            + [pltpu.VMEM((128,), jnp.float32),
               pltpu.VMEM((ROWS_PER_TILE,), jnp.float32),
               pltpu.VMEM_SHARED((NP, H), jnp.float32),
               pltpu.VMEM_SHARED((NP,), jnp.float32),
               pltpu.SemaphoreType.DMA,
               pltpu.SemaphoreType.DMA,
               pltpu.SemaphoreType.DMA]
        ),
    )


def _gat_body(table_hbm, s_hbm, d_hbm, src_hbm, dst_hbm, zeros_hbm,
              out_hbm, den_hbm, src_v, dst_v, r0_, r1_, r2_, r3_,
              s_t, d_t, ex_vs, zb, acc_sh, den_sh, semg, sems):
    c = lax.axis_index("c")
    s = lax.axis_index("s")
    w = c * NS + s
    rows = [r0_, r1_, r2_, r3_]
    r = pl.ds(s * ROWS_PER_TILE, ROWS_PER_TILE)
    pltpu.sync_copy(zeros_hbm.at[r], acc_sh.at[r])
    for i in range(ROWS_PER_TILE // 16):
        zb[pl.ds(i * 16, 16)] = jnp.zeros((16,), jnp.float32)
    pltpu.sync_copy(zb, den_sh.at[r])
    pltpu.sync_copy(s_hbm, s_t)
    pltpu.sync_copy(d_hbm, d_t)
    plsc.subcore_barrier()
    _gat_phase(w, NBLK_SL, table_hbm, src_hbm, dst_hbm, acc_sh, den_sh,
               src_v, dst_v, rows, s_t, d_t, ex_vs, semg, sems)
    plsc.subcore_barrier()
    pltpu.sync_copy(acc_sh.at[r], out_hbm.at[c, r])
    pltpu.sync_copy(den_sh.at[r], den_hbm.at[c, r])


def _make_gat():
    return pl.kernel(
        _gat_body,
        out_type=(jax.ShapeDtypeStruct((NC, NP, H), jnp.float32),
                  jax.ShapeDtypeStruct((NC, NP), jnp.float32)),
        mesh=_mesh,
        compiler_params=pltpu.CompilerParams(needs_layout_passes=False, use_tc_tiling_on_sc=False),
        scratch_types=(
            [pltpu.VMEM((GPB, 128), jnp.int32)] * 2
            + [pltpu.VMEM((128, H), jnp.float32)] * 4
            + [pltpu.VMEM((NP,), jnp.float32),
               pltpu.VMEM((NP,), jnp.float32),
               pltpu.VMEM((GPB, 128), jnp.float32),
               pltpu.VMEM((ROWS_PER_TILE,), jnp.float32),
               pltpu.VMEM_SHARED((NP, H), jnp.float32),
               pltpu.VMEM_SHARED((NP,), jnp.float32),
               pltpu.SemaphoreType.DMA,
               pltpu.SemaphoreType.DMA]
        ),
    )


def _rowpass_body(nblk, table_hbm, src_hbm, dst_hbm, zeros_hbm, out_hbm,
                  s0, s1, s2, s3, d0, d1, d2, d3,
                  r0_, r1_, r2_, r3_, r4_, r5_, r6_, r7_, acc_sh,
                  semi, semg, sems):
    c = lax.axis_index("c")
    s = lax.axis_index("s")
    w = c * NS + s
    srcv = [s0, s1, s2, s3]
    dstv = [d0, d1, d2, d3]
    rows = [r0_, r1_, r2_, r3_, r4_, r5_, r6_, r7_]
    r = pl.ds(s * ROWS_PER_TILE, ROWS_PER_TILE)
    pltpu.sync_copy(zeros_hbm.at[r], acc_sh.at[r])
    plsc.subcore_barrier()
    _stream_phase(w, nblk, False, table_hbm, src_hbm, dst_hbm,
                  acc_sh, None, None, srcv, dstv, rows, semi, semg, sems)
    plsc.subcore_barrier()
    pltpu.sync_copy(acc_sh.at[r], out_hbm.at[c, r])


def _make_rowpass(n_edges_pad):
    nblk = n_edges_pad // (NW * BLK)
    return pl.kernel(
        functools.partial(_rowpass_body, nblk),
        out_type=jax.ShapeDtypeStruct((NC, NP, H), jnp.float32),
        mesh=_mesh,
        compiler_params=pltpu.CompilerParams(needs_layout_passes=False, use_tc_tiling_on_sc=False),
        scratch_types=(
            [pltpu.VMEM((GPB, 128), jnp.int32)] * 8
            + [pltpu.VMEM((128, H), jnp.float32)] * 8
            + [pltpu.VMEM_SHARED((NP, H), jnp.float32),
               pltpu.SemaphoreType.DMA,
               pltpu.SemaphoreType.DMA,
               pltpu.SemaphoreType.DMA]
        ),
    )


RB = 1280
NG = NP // RB


def _tc1_body(xp_ref, wcat_ref, a1_ref,
              tg_ref, ta_ref, ts_ref, xr_ref, sd_ref):
    xb = xp_ref[...]
    h4 = jnp.dot(xb, wcat_ref[...], preferred_element_type=jnp.float32)
    tg_ref[...] = h4[:, 0:H]
    ta = h4[:, H:2 * H]
    ta_ref[...] = ta
    ts_ref[...] = h4[:, 2 * H:3 * H]
    xr_ref[...] = h4[:, 3 * H:4 * H]
    sd_ref[...] = jnp.dot(ta, a1_ref[...], preferred_element_type=jnp.float32)


def _dinv_body(t_ref, degT_ref, out_ref):
    deg = degT_ref[:, 0:1] + degT_ref[:, 1:2]
    out_ref[...] = t_ref[...] * lax.rsqrt(deg + 1.0)


def _dinv_scale(t, degT):
    f = pl.pallas_call(
        _dinv_body,
        grid=(NG,),
        in_specs=[
            pl.BlockSpec((RB, H), lambda i: (i, 0)),
            pl.BlockSpec((RB, 2), lambda i: (i, 0)),
        ],
        out_specs=pl.BlockSpec((RB, H), lambda i: (i, 0)),
        out_shape=jax.ShapeDtypeStruct((NP, H), jnp.float32),
    )
    return f(t, degT)


def _tc1(xp, wcat, a1):
    f = pl.pallas_call(
        _tc1_body,
        grid=(NG,),
        in_specs=[
            pl.BlockSpec((RB, D), lambda i: (i, 0)),
            pl.BlockSpec((D, 4 * H), lambda i: (0, 0)),
            pl.BlockSpec((H, 128), lambda i: (0, 0)),
        ],
        out_specs=[
            pl.BlockSpec((RB, H), lambda i: (i, 0)),
            pl.BlockSpec((RB, H), lambda i: (i, 0)),
            pl.BlockSpec((RB, H), lambda i: (i, 0)),
            pl.BlockSpec((RB, H), lambda i: (i, 0)),
            pl.BlockSpec((RB, 128), lambda i: (i, 0)),
        ],
        out_shape=[jax.ShapeDtypeStruct((NP, H), jnp.float32)] * 4
        + [jax.ShapeDtypeStruct((NP, 128), jnp.float32)],
    )
    return f(xp, wcat, a1)


def _tc2_body(accg_ref, acca_ref, accs_ref, denT_ref, degT_ref, xr1_ref,
              wg_ref, wa_ref, wslr_ref, a2_ref, b1_ref,
              tg2_ref, ta2_ref, ts2_ref, xr2_ref, sd2_ref):
    deg = degT_ref[:, 0:1] + degT_ref[:, 1:2]
    dinv = lax.rsqrt(deg + 1.0)
    den = denT_ref[:, 0:1] + denT_ref[:, 1:2]
    h1g = jnp.maximum(dinv * (accg_ref[0] + accg_ref[1]) + b1_ref[0:1, :], 0.0)
    h1a = jnp.maximum((acca_ref[0] + acca_ref[1]) / den + b1_ref[1:2, :], 0.0)
    h1s = jnp.maximum((accs_ref[0] + accs_ref[1]) / jnp.maximum(deg, 1.0)
                      + xr1_ref[...] + b1_ref[2:3, :], 0.0)
    tg2_ref[...] = dinv * jnp.dot(h1g, wg_ref[...],
                                  preferred_element_type=jnp.float32)
    ta2 = jnp.dot(h1a, wa_ref[...], preferred_element_type=jnp.float32)
    ta2_ref[...] = ta2
    hs2 = jnp.dot(h1s, wslr_ref[...], preferred_element_type=jnp.float32)
    ts2_ref[...] = hs2[:, 0:H]
    xr2_ref[...] = hs2[:, H:2 * H]
    sd2_ref[...] = jnp.dot(ta2, a2_ref[...], preferred_element_type=jnp.float32)


def _tc2(accg, acca, accs, denT, degT, xr1, wg, wa, wslr, a2, b1):
    f = pl.pallas_call(
        _tc2_body,
        grid=(NG,),
        in_specs=[
            pl.BlockSpec((NC, RB, H), lambda i: (0, i, 0)),
            pl.BlockSpec((NC, RB, H), lambda i: (0, i, 0)),
            pl.BlockSpec((NC, RB, H), lambda i: (0, i, 0)),
            pl.BlockSpec((RB, 2), lambda i: (i, 0)),
            pl.BlockSpec((RB, 2), lambda i: (i, 0)),
            pl.BlockSpec((RB, H), lambda i: (i, 0)),
            pl.BlockSpec((H, H), lambda i: (0, 0)),
            pl.BlockSpec((H, H), lambda i: (0, 0)),
            pl.BlockSpec((H, 2 * H), lambda i: (0, 0)),
            pl.BlockSpec((H, 128), lambda i: (0, 0)),
            pl.BlockSpec((3, H), lambda i: (0, 0)),
        ],
        out_specs=[
            pl.BlockSpec((RB, H), lambda i: (i, 0)),
            pl.BlockSpec((RB, H), lambda i: (i, 0)),
            pl.BlockSpec((RB, H), lambda i: (i, 0)),
            pl.BlockSpec((RB, H), lambda i: (i, 0)),
            pl.BlockSpec((RB, 128), lambda i: (i, 0)),
        ],
        out_shape=[jax.ShapeDtypeStruct((NP, H), jnp.float32)] * 4
        + [jax.ShapeDtypeStruct((NP, 128), jnp.float32)],
    )
    return f(accg, acca, accs, denT, degT, xr1, wg, wa, wslr, a2, b1)


def _tc3_body(accg_ref, acca_ref, accs_ref, denT_ref, degT_ref, xr2_ref,
              batchT_ref, b2_ref, wfc_ref, fcb_ref, out_ref,
              pooled_ref, cnt_ref):
    i = pl.program_id(0)

    @pl.when(i == 0)
    def _():
        pooled_ref[...] = jnp.zeros_like(pooled_ref)
        cnt_ref[...] = jnp.zeros_like(cnt_ref)

    deg = degT_ref[:, 0:1] + degT_ref[:, 1:2]
    dinv = lax.rsqrt(deg + 1.0)
    den = denT_ref[:, 0:1] + denT_ref[:, 1:2]
    h2g = jnp.maximum(dinv * (accg_ref[0] + accg_ref[1]) + b2_ref[0:1, :], 0.0)
    h2a = jnp.maximum((acca_ref[0] + acca_ref[1]) / den + b2_ref[1:2, :], 0.0)
    h2s = jnp.maximum((accs_ref[0] + accs_ref[1]) / jnp.maximum(deg, 1.0)
                      + xr2_ref[...] + b2_ref[2:3, :], 0.0)
    hcat = jnp.concatenate([h2g, h2a, h2s], axis=1)
    pb = (batchT_ref[...] ==
          lax.broadcasted_iota(jnp.int32, (G, RB), 0)).astype(jnp.float32)
    pooled_ref[...] += jnp.dot(pb, hcat, preferred_element_type=jnp.float32)
    cnt_ref[...] += jnp.sum(pb, axis=1, keepdims=True)

    @pl.when(i == NG - 1)
    def _():
        cnt = jnp.maximum(cnt_ref[...], 1.0)
        pool = pooled_ref[...] / cnt
        out_ref[...] = (jnp.dot(pool, wfc_ref[...],
                                preferred_element_type=jnp.float32)
                        + fcb_ref[...]) * (1.0 / 3.0)


def _tc3(accg, acca, accs, denT, degT, xr2, batchT, b2, wfc, fcb):
    f = pl.pallas_call(
        _tc3_body,
        grid=(NG,),
        in_specs=[
            pl.BlockSpec((NC, RB, H), lambda i: (0, i, 0)),
            pl.BlockSpec((NC, RB, H), lambda i: (0, i, 0)),
            pl.BlockSpec((NC, RB, H), lambda i: (0, i, 0)),
            pl.BlockSpec((RB, 2), lambda i: (i, 0)),
            pl.BlockSpec((RB, 2), lambda i: (i, 0)),
            pl.BlockSpec((RB, H), lambda i: (i, 0)),
            pl.BlockSpec((1, RB), lambda i: (0, i)),
            pl.BlockSpec((3, H), lambda i: (0, 0)),
            pl.BlockSpec((3 * H, 128), lambda i: (0, 0)),
            pl.BlockSpec((1, 128), lambda i: (0, 0)),
        ],
        out_specs=pl.BlockSpec((G, 128), lambda i: (0, 0)),
        out_shape=jax.ShapeDtypeStruct((G, 128), jnp.float32),
        scratch_shapes=[
            pltpu.VMEM((G, 3 * H), jnp.float32),
            pltpu.VMEM((G, 1), jnp.float32),
        ],
    )
    return f(accg, acca, accs, denT, degT, xr2, batchT, b2, wfc, fcb)


def kernel(x, edge_index, batch, params):
    p = params
    src = edge_index[0].astype(jnp.int32)
    dst = edge_index[1].astype(jnp.int32)
    loop = jnp.arange(N, dtype=jnp.int32)

    npad_sl = ESL - (320000 + N)
    pad_i_sl = jnp.arange(npad_sl, dtype=jnp.int32)
    src_sl = jnp.concatenate([src, loop, pad_i_sl % 64]).reshape(ESL // 128, 128)
    dst_sl = jnp.concatenate([dst, loop, N + pad_i_sl % (NP - N)]
                             ).reshape(ESL // 128, 128)
    npad_sg = ESG - 320000
    pad_i_sg = jnp.arange(npad_sg, dtype=jnp.int32)
    src_sg = jnp.concatenate([src, pad_i_sg % 64]).reshape(ESG // 128, 128)
    dst_sg = jnp.concatenate([dst, N + pad_i_sg % (NP - N)]
                             ).reshape(ESG // 128, 128)

    xp = jnp.pad(x, ((0, NP - N), (0, 0)))
    zeros = jnp.zeros((NP, H), jnp.float32)
    batchT = jnp.pad(batch.astype(jnp.int32), (0, NP - N),
                     constant_values=-1).reshape(1, NP)

    wcat1 = jnp.concatenate(
        [p['gcn_W1'], p['gat_W1'], p['sage_W1l'], p['sage_W1r']], axis=1)
    a1 = jnp.zeros((H, 128), jnp.float32)
    a1 = a1.at[:, 0].set(p['gat_as1']).at[:, 1].set(p['gat_ad1'])
    a2 = jnp.zeros((H, 128), jnp.float32)
    a2 = a2.at[:, 0].set(p['gat_as2']).at[:, 1].set(p['gat_ad2'])
    wslr2 = jnp.concatenate([p['sage_W2l'], p['sage_W2r']], axis=1)
    b1 = jnp.stack([p['gcn_b1'], p['gat_b1'], p['sage_b1']])
    b2 = jnp.stack([p['gcn_b2'], p['gat_b2'], p['sage_b2']])
    wfc = jnp.zeros((3 * H, 128), jnp.float32)
    wfc = wfc.at[0:H, 0:C].set(p['gcn_fcW'])
    wfc = wfc.at[H:2 * H, 0:C].set(p['gat_fcW'])
    wfc = wfc.at[2 * H:3 * H, 0:C].set(p['sage_fcW'])
    fcb = jnp.zeros((1, 128), jnp.float32)
    fcb = fcb.at[0, 0:C].set(p['gcn_fcb'] + p['gat_fcb'] + p['sage_fcb'])

    tg1r, ta1, ts1, xr1, sd1 = _tc1(xp, wcat1, a1)
    s1 = sd1[:, 0] + 0.0
    d1 = sd1[:, 1] + 0.0

    rowpass_sl = _make_rowpass(ESL)
    rowpass_sg = _make_rowpass(ESG)
    sagedeg = _make_sagedeg()
    gat = _make_gat()

    accs1, degp = sagedeg(ts1, src_sg, dst_sg, zeros)
    degT = degp.T
    acca1, den1 = gat(ta1, s1, d1, src_sl, dst_sl, zeros)
    tg1 = _dinv_scale(tg1r, degT)
    accg1 = rowpass_sl(tg1, src_sl, dst_sl, zeros)

    tg2, ta2, ts2, xr2, sd2 = _tc2(accg1, acca1, accs1, den1.T, degT, xr1,
                                   p['gcn_W2'], p['gat_W2'], wslr2, a2, b1)
    s2 = sd2[:, 0] + 0.0
    d2 = sd2[:, 1] + 0.0

    accg2 = rowpass_sl(tg2, src_sl, dst_sl, zeros)
    acca2, den2 = gat(ta2, s2, d2, src_sl, dst_sl, zeros)
    accs2 = rowpass_sg(ts2, src_sg, dst_sg, zeros)

    out128 = _tc3(accg2, acca2, accs2, den2.T, degT, xr2, batchT, b2, wfc, fcb)
    return out128[:, :C]
